# Initial kernel scaffold; baseline (speedup 1.0000x reference)
import jax
import jax.numpy as jnp
from jax.experimental import pallas as pl


def kernel(x, edge_index, batch, fc0_w, fc0_b, conv1_weight, conv1_u, conv1_c, conv1_bias, conv2_weight, conv2_u, conv2_c, conv2_bias, fc1_w, fc1_b):
    raise NotImplementedError("write your pallas kernel here")


# speedup vs baseline: 2.7770x; 2.7770x over previous
import functools

import jax
import jax.numpy as jnp
from jax import lax
from jax.experimental import pallas as pl
from jax.experimental.pallas import tpu as pltpu
from jax.experimental.pallas import tpu_sc as plsc

N = 50000
E = 800000
HEADS = 8
BLK = 1000
NBLK = N // BLK

SC_CORES = 2
SC_TILES = 16
SC_WORKERS = SC_CORES * SC_TILES
CHUNK = 64
CHUNKS_PER_TILE = -(-E // (SC_WORKERS * CHUNK))
EPAD = SC_WORKERS * CHUNK * CHUNKS_PER_TILE
ACC_ROWS = 50176
ZCH = 64
ZN = ACC_ROWS // SC_TILES // ZCH
HU_ROWS = 50016
DUMMY_DST = N + 8


def _dense1_body(x_ref, w0_ref, b0_ref, w1_ref, u1_ref, q0_ref,
                 hw_ref, hu_ref, sm_ref):
    h = jnp.maximum(jnp.dot(x_ref[...], w0_ref[...],
                            preferred_element_type=jnp.float32)
                    + b0_ref[...], 0.0)
    hw = jnp.dot(h, w1_ref[...], preferred_element_type=jnp.float32)
    hw_ref[...] = hw
    hu = jnp.dot(h, u1_ref[...], preferred_element_type=jnp.float32)
    hu_ref[...] = jnp.concatenate([hu, jnp.zeros_like(hu)], axis=1)
    q0 = q0_ref[...]
    sm = jnp.zeros((x_ref.shape[0], 32), jnp.float32)
    for hh in range(HEADS):
        sm = sm + q0[0:1, hh:hh + 1] * hw[:, 32 * hh:32 * hh + 32]
    sm_ref[...] = sm


def _dense1(x, w0, b0, w1, u1, q0):
    return pl.pallas_call(
        _dense1_body,
        grid=(NBLK,),
        in_specs=[
            pl.BlockSpec((BLK, 128), lambda i: (i, 0)),
            pl.BlockSpec((128, 16), lambda i: (0, 0)),
            pl.BlockSpec((1, 16), lambda i: (0, 0)),
            pl.BlockSpec((16, 256), lambda i: (0, 0)),
            pl.BlockSpec((16, 8), lambda i: (0, 0)),
            pl.BlockSpec((1, 8), lambda i: (0, 0)),
        ],
        out_specs=[
            pl.BlockSpec((BLK, 256), lambda i: (i, 0)),
            pl.BlockSpec((BLK, 16), lambda i: (i, 0)),
            pl.BlockSpec((BLK, 32), lambda i: (i, 0)),
        ],
        out_shape=[
            jax.ShapeDtypeStruct((N, 256), jnp.float32),
            jax.ShapeDtypeStruct((N, 16), jnp.float32),
            jax.ShapeDtypeStruct((N, 32), jnp.float32),
        ],
    )(x, w0, b0, w1, u1, q0)


def _dense2_body(acc_ref, dg_ref, sm1_ref, b1_ref, w2_ref, u2_ref, q0_ref,
                 ta_ref, tb_ref, hu_ref, sm2_ref, deg_ref):
    a = acc_ref[0] + acc_ref[1] + sm1_ref[...]
    deg = dg_ref[0, :, 0:1] + dg_ref[1, :, 0:1] + 1.0
    deg_ref[...] = deg
    out1 = jnp.maximum(a / deg + b1_ref[...], 0.0)
    hw2 = jnp.dot(out1, w2_ref[...], preferred_element_type=jnp.float32)
    q0 = q0_ref[...]
    sm = jnp.zeros((out1.shape[0], 64), jnp.float32)
    for hh in range(HEADS):
        blk = hw2[:, 64 * hh:64 * hh + 64]
        sm = sm + q0[0:1, hh:hh + 1] * blk
        ta_ref[:, 32 * hh:32 * hh + 32] = blk[:, 0:32]
        tb_ref[:, 32 * hh:32 * hh + 32] = blk[:, 32:64]
    sm2_ref[...] = sm
    hu = jnp.dot(out1, u2_ref[...], preferred_element_type=jnp.float32)
    hu_ref[...] = jnp.concatenate([hu, jnp.zeros_like(hu)], axis=1)


def _dense2(acc1, dega, sm1, b1, w2, u2, q0):
    return pl.pallas_call(
        _dense2_body,
        grid=(NBLK,),
        in_specs=[
            pl.BlockSpec((2, BLK, 32), lambda i: (0, i, 0)),
            pl.BlockSpec((2, BLK, 16), lambda i: (0, i, 0)),
            pl.BlockSpec((BLK, 32), lambda i: (i, 0)),
            pl.BlockSpec((1, 32), lambda i: (0, 0)),
            pl.BlockSpec((32, 512), lambda i: (0, 0)),
            pl.BlockSpec((32, 8), lambda i: (0, 0)),
            pl.BlockSpec((1, 8), lambda i: (0, 0)),
        ],
        out_specs=[
            pl.BlockSpec((BLK, 256), lambda i: (i, 0)),
            pl.BlockSpec((BLK, 256), lambda i: (i, 0)),
            pl.BlockSpec((BLK, 16), lambda i: (i, 0)),
            pl.BlockSpec((BLK, 64), lambda i: (i, 0)),
            pl.BlockSpec((BLK, 1), lambda i: (i, 0)),
        ],
        out_shape=[
            jax.ShapeDtypeStruct((N, 256), jnp.float32),
            jax.ShapeDtypeStruct((N, 256), jnp.float32),
            jax.ShapeDtypeStruct((N, 16), jnp.float32),
            jax.ShapeDtypeStruct((N, 64), jnp.float32),
            jax.ShapeDtypeStruct((N, 1), jnp.float32),
        ],
    )(acc1, dega, sm1, b1, w2, u2, q0)


def _dense3_body(aa_ref, ab_ref, sm2_ref, deg_ref, b2_ref, oh_ref,
                 w3_ref, b3_ref, out_ref, sums_ref, cnt_ref):
    i = pl.program_id(0)

    @pl.when(i == 0)
    def _():
        sums_ref[...] = jnp.zeros_like(sums_ref)
        cnt_ref[...] = jnp.zeros_like(cnt_ref)

    deg = deg_ref[...]
    a = (aa_ref[0] + aa_ref[1] + sm2_ref[:, 0:32]) / deg
    b = (ab_ref[0] + ab_ref[1] + sm2_ref[:, 32:64]) / deg
    out2 = jnp.maximum(jnp.concatenate([a, b], axis=1) + b2_ref[...], 0.0)
    oh = oh_ref[...]
    sums_ref[...] += lax.dot_general(oh, out2, (((0,), (0,)), ((), ())),
                                     preferred_element_type=jnp.float32)
    cnt_ref[...] += lax.dot_general(oh, jnp.ones_like(oh),
                                    (((0,), (0,)), ((), ())),
                                    preferred_element_type=jnp.float32)

    @pl.when(i == NBLK - 1)
    def _():
        g = sums_ref[...] / jnp.maximum(cnt_ref[:, 0:1], 1.0)
        out_ref[...] = jnp.dot(g, w3_ref[...],
                               preferred_element_type=jnp.float32) + b3_ref[...]


def _dense3(accA, accB, sm2, deg, b2, oh, w3, b3):
    return pl.pallas_call(
        _dense3_body,
        grid=(NBLK,),
        in_specs=[
            pl.BlockSpec((2, BLK, 32), lambda i: (0, i, 0)),
            pl.BlockSpec((2, BLK, 32), lambda i: (0, i, 0)),
            pl.BlockSpec((BLK, 64), lambda i: (i, 0)),
            pl.BlockSpec((BLK, 1), lambda i: (i, 0)),
            pl.BlockSpec((1, 64), lambda i: (0, 0)),
            pl.BlockSpec((BLK, 16), lambda i: (i, 0)),
            pl.BlockSpec((64, 16), lambda i: (0, 0)),
            pl.BlockSpec((1, 16), lambda i: (0, 0)),
        ],
        out_specs=pl.BlockSpec((16, 16), lambda i: (0, 0)),
        out_shape=jax.ShapeDtypeStruct((16, 16), jnp.float32),
        scratch_shapes=[
            pltpu.VMEM((16, 64), jnp.float32),
            pltpu.VMEM((16, 16), jnp.float32),
        ],
    )(accA, accB, sm2, deg, b2, oh, w3, b3)


_SC_MESH = plsc.VectorSubcoreMesh(core_axis_name="c", subcore_axis_name="s")
_ROWS_PER_TILE = ACC_ROWS // SC_TILES
_TAKE_DN = lax.GatherDimensionNumbers(
    offset_dims=(), collapsed_slice_dims=(0,), start_index_map=(0,))


def _take(v, idx):
    return lax.gather(v, idx[:, None], _TAKE_DN, (1,),
                      mode=lax.GatherScatterMode.PROMISE_IN_BOUNDS)


@functools.partial(
    pl.kernel,
    out_type=jax.ShapeDtypeStruct((SC_CORES * ACC_ROWS, 32), jnp.float32),
    mesh=_SC_MESH,
    scratch_types=[
        pltpu.VMEM((CHUNK,), jnp.int32),
        pltpu.VMEM((CHUNK,), jnp.int32),
        pltpu.VMEM((CHUNK, 16), jnp.float32),
        pltpu.VMEM((CHUNK, 16), jnp.float32),
        pltpu.VMEM((CHUNK, 256), jnp.float32),
        pltpu.VMEM((CHUNK, 32), jnp.float32),
        pltpu.VMEM((16,), jnp.float32),
        pltpu.VMEM_SHARED((ACC_ROWS, 32), jnp.float32),
        pltpu.SemaphoreType.DMA,
        pltpu.SemaphoreType.DMA,
        pltpu.SemaphoreType.DMA,
    ],
    compiler_params=pltpu.CompilerParams(use_tc_tiling_on_sc=False),
)
def _edge_sc(src_hbm, dst_hbm, hu_hbm, tab_hbm, cc_hbm, out_hbm,
             sidx, didx, hus, hud, rows, msg, ccv, acc, sem1, sem2, sem3):
    cid = lax.axis_index("c")
    sid = lax.axis_index("s")
    wid = sid * SC_CORES + cid
    iota = lax.iota(jnp.int32, 16)

    zv = jnp.zeros((16,), jnp.float32)

    def _zrow(i, _):
        msg[i, pl.ds(0, 16)] = zv
        msg[i, pl.ds(16, 16)] = zv
        return 0

    lax.fori_loop(0, CHUNK, _zrow, 0)

    def _zacc(z, _):
        pltpu.sync_copy(msg, acc.at[pl.ds(sid * _ROWS_PER_TILE + z * ZCH,
                                          ZCH)])
        return 0

    lax.fori_loop(0, ZN, _zacc, 0)

    pltpu.sync_copy(cc_hbm, ccv)
    cc = ccv[...]
    r8 = (iota + 8) & 15
    r4 = (iota + 4) & 15
    r2 = (iota + 2) & 15
    r1 = (iota + 1) & 15
    plsc.subcore_barrier()

    tile_base = wid * (CHUNKS_PER_TILE * CHUNK)

    def _chunk(g, _):
        base = tile_base + g * CHUNK
        pltpu.sync_copy(src_hbm.at[pl.ds(base, CHUNK)], sidx)
        pltpu.sync_copy(dst_hbm.at[pl.ds(base, CHUNK)], didx)
        cp1 = pltpu.async_copy(hu_hbm.at[sidx], hus, sem1)
        cp2 = pltpu.async_copy(hu_hbm.at[didx], hud, sem2)
        cp3 = pltpu.async_copy(tab_hbm.at[sidx], rows, sem3)
        cp1.wait()
        cp2.wait()
        cp3.wait()

        def _edge(e, _):
            t = jnp.exp(hus[e] - hud[e] + cc)
            s = t + _take(t, r8)
            s = s + _take(s, r4)
            s = s + _take(s, r2)
            s = s + _take(s, r1)
            q = t / s
            qh = _take(q, jnp.zeros((16,), jnp.int32))
            m0 = qh * rows[e, pl.ds(0, 16)]
            m1 = qh * rows[e, pl.ds(16, 16)]
            for hh in range(1, HEADS):
                qh = _take(q, jnp.full((16,), hh, jnp.int32))
                m0 = m0 + qh * rows[e, pl.ds(32 * hh, 16)]
                m1 = m1 + qh * rows[e, pl.ds(32 * hh + 16, 16)]
            msg[e, pl.ds(0, 16)] = m0
            msg[e, pl.ds(16, 16)] = m1
            return 0

        lax.fori_loop(0, CHUNK, _edge, 0)
        pltpu.sync_copy(msg, acc.at[didx], add=True)
        return 0

    lax.fori_loop(0, CHUNKS_PER_TILE, _chunk, 0)
    plsc.subcore_barrier()

    def _copyout(z, _):
        r0 = sid * _ROWS_PER_TILE + z * ZCH
        pltpu.sync_copy(acc.at[pl.ds(r0, ZCH)],
                        out_hbm.at[pl.ds(cid * ACC_ROWS + r0, ZCH)])
        return 0

    lax.fori_loop(0, ZN, _copyout, 0)


@functools.partial(
    pl.kernel,
    out_type=jax.ShapeDtypeStruct((SC_CORES * ACC_ROWS, 16), jnp.float32),
    mesh=_SC_MESH,
    scratch_types=[
        pltpu.VMEM((CHUNK,), jnp.int32),
        pltpu.VMEM((CHUNK, 16), jnp.float32),
        pltpu.VMEM((ZCH, 16), jnp.float32),
        pltpu.VMEM_SHARED((ACC_ROWS, 16), jnp.float32),
    ],
    compiler_params=pltpu.CompilerParams(use_tc_tiling_on_sc=False),
)
def _deg_sc(dst_hbm, out_hbm, didx, ones, zrows, acc):
    cid = lax.axis_index("c")
    sid = lax.axis_index("s")
    wid = sid * SC_CORES + cid
    iota = lax.iota(jnp.int32, 16)
    onev = jnp.where(iota == 0, 1.0, 0.0).astype(jnp.float32)
    zv = jnp.zeros((16,), jnp.float32)

    def _fill(i, _):
        ones[i, pl.ds(0, 16)] = onev
        zrows[i, pl.ds(0, 16)] = zv
        return 0

    lax.fori_loop(0, max(CHUNK, ZCH), _fill, 0)

    def _zacc(z, _):
        pltpu.sync_copy(zrows, acc.at[pl.ds(sid * _ROWS_PER_TILE + z * ZCH,
                                            ZCH)])
        return 0

    lax.fori_loop(0, ZN, _zacc, 0)
    plsc.subcore_barrier()

    tile_base = wid * (CHUNKS_PER_TILE * CHUNK)

    def _chunk(g, _):
        pltpu.sync_copy(dst_hbm.at[pl.ds(tile_base + g * CHUNK, CHUNK)], didx)
        pltpu.sync_copy(ones, acc.at[didx], add=True)
        return 0

    lax.fori_loop(0, CHUNKS_PER_TILE, _chunk, 0)
    plsc.subcore_barrier()

    def _copyout(z, _):
        r0 = sid * _ROWS_PER_TILE + z * ZCH
        pltpu.sync_copy(acc.at[pl.ds(r0, ZCH)],
                        out_hbm.at[pl.ds(cid * ACC_ROWS + r0, ZCH)])
        return 0

    lax.fori_loop(0, ZN, _copyout, 0)


def _edge_jnp(src, dst, hu_pad, table, cc, msg_w):
    hs = hu_pad[src][:, 0:8]
    hd = hu_pad[dst][:, 0:8]
    q = jax.nn.softmax(hs - hd + cc[None, 0:8], axis=-1)
    rows = table[src]
    msg = jnp.zeros((src.shape[0], 32), jnp.float32)
    for hh in range(HEADS):
        msg = msg + q[:, hh:hh + 1] * rows[:, 32 * hh:32 * hh + 32]
    if msg_w == 40:
        msg = jnp.concatenate(
            [msg, jnp.ones((src.shape[0], 1), jnp.float32),
             jnp.zeros((src.shape[0], 7), jnp.float32)], axis=1)
    acc = jax.ops.segment_sum(msg, dst, num_segments=ACC_ROWS)
    return jnp.stack([acc, jnp.zeros_like(acc)])


def kernel(x, edge_index, batch, fc0_w, fc0_b, conv1_weight, conv1_u, conv1_c,
           conv1_bias, conv2_weight, conv2_u, conv2_c, conv2_bias, fc1_w,
           fc1_b):
    f32 = jnp.float32
    q01 = jax.nn.softmax(conv1_c)[None, :]
    q02 = jax.nn.softmax(conv2_c)[None, :]
    cc1 = jnp.concatenate([conv1_c, jnp.full((8,), -1e30, f32)])
    cc2 = jnp.concatenate([conv2_c, jnp.full((8,), -1e30, f32)])

    src = jnp.concatenate(
        [edge_index[0], jnp.zeros((EPAD - E,), jnp.int32)])
    dst = jnp.concatenate(
        [edge_index[1], jnp.full((EPAD - E,), DUMMY_DST, jnp.int32)])

    hw1, hu1, sm1 = _dense1(x, fc0_w, fc0_b[None, :], conv1_weight, conv1_u,
                            q01)
    hu1p = jnp.zeros((HU_ROWS, 16), f32).at[0:N].set(hu1)
    dega = _deg_sc(dst).reshape(2, ACC_ROWS, 16)
    acc1 = _edge_sc(src, dst, hu1p, hw1, cc1).reshape(2, ACC_ROWS, 32)

    ta, tb, hu2, sm2, deg = _dense2(acc1, dega, sm1, conv1_bias[None, :],
                                    conv2_weight, conv2_u, q02)
    hu2p = jnp.zeros((HU_ROWS, 16), f32).at[0:N].set(hu2)
    accA = _edge_sc(src, dst, hu2p, ta, cc2).reshape(2, ACC_ROWS, 32)
    accB = _edge_sc(src, dst, hu2p, tb, cc2).reshape(2, ACC_ROWS, 32)

    oh = (batch[:, None] == jnp.arange(16, dtype=jnp.int32)[None, :]
          ).astype(f32)
    out16 = _dense3(accA, accB, sm2, deg, conv2_bias[None, :], oh,
                    jnp.zeros((64, 16), f32).at[:, 0:10].set(fc1_w),
                    jnp.zeros((1, 16), f32).at[0, 0:10].set(fc1_b))
    return out16[:, 0:10]


# speedup vs baseline: 4.1395x; 1.4906x over previous
import functools

import jax
import jax.numpy as jnp
from jax import lax
from jax.experimental import pallas as pl
from jax.experimental.pallas import tpu as pltpu
from jax.experimental.pallas import tpu_sc as plsc

N = 50000
E = 800000
HEADS = 8
BLK = 1000
NBLK = N // BLK

SC_CORES = 2
SC_TILES = 16
SC_WORKERS = SC_CORES * SC_TILES
CHUNK = 32
NCHUNK = 784
EPAD = SC_WORKERS * CHUNK * NCHUNK
DCH = 128
DNCH = EPAD // (SC_WORKERS * DCH)
ACC_ROWS = 50176
ZCH = 64
ZN = ACC_ROWS // SC_TILES // ZCH
HU_ROWS = 50016
DUMMY_DST = N + 8


def _dense1_body(x_ref, w0_ref, b0_ref, w1_ref, u1_ref, q0_ref,
                 hw_ref, hu_ref, sm_ref):
    h = jnp.maximum(jnp.dot(x_ref[...], w0_ref[...],
                            preferred_element_type=jnp.float32)
                    + b0_ref[...], 0.0)
    hw = jnp.dot(h, w1_ref[...], preferred_element_type=jnp.float32)
    hw_ref[...] = hw
    hu = jnp.dot(h, u1_ref[...], preferred_element_type=jnp.float32)
    hu_ref[...] = jnp.concatenate([hu, jnp.zeros_like(hu)], axis=1)
    q0 = q0_ref[...]
    sm = jnp.zeros((x_ref.shape[0], 32), jnp.float32)
    for hh in range(HEADS):
        sm = sm + q0[0:1, hh:hh + 1] * hw[:, 32 * hh:32 * hh + 32]
    sm_ref[...] = sm


def _dense1(x, w0, b0, w1, u1, q0):
    return pl.pallas_call(
        _dense1_body,
        grid=(NBLK,),
        in_specs=[
            pl.BlockSpec((BLK, 128), lambda i: (i, 0)),
            pl.BlockSpec((128, 16), lambda i: (0, 0)),
            pl.BlockSpec((1, 16), lambda i: (0, 0)),
            pl.BlockSpec((16, 256), lambda i: (0, 0)),
            pl.BlockSpec((16, 8), lambda i: (0, 0)),
            pl.BlockSpec((1, 8), lambda i: (0, 0)),
        ],
        out_specs=[
            pl.BlockSpec((BLK, 256), lambda i: (i, 0)),
            pl.BlockSpec((BLK, 16), lambda i: (i, 0)),
            pl.BlockSpec((BLK, 32), lambda i: (i, 0)),
        ],
        out_shape=[
            jax.ShapeDtypeStruct((N, 256), jnp.float32),
            jax.ShapeDtypeStruct((N, 16), jnp.float32),
            jax.ShapeDtypeStruct((N, 32), jnp.float32),
        ],
    )(x, w0, b0, w1, u1, q0)


def _dense2_body(acc_ref, dg_ref, sm1_ref, b1_ref, w2_ref, u2_ref, q0_ref,
                 ta_ref, tb_ref, hu_ref, sm2_ref, deg_ref):
    a = acc_ref[0] + acc_ref[1] + sm1_ref[...]
    deg = dg_ref[0, :, 0:1] + dg_ref[1, :, 0:1] + 1.0
    deg_ref[...] = deg
    out1 = jnp.maximum(a / deg + b1_ref[...], 0.0)
    hw2 = jnp.dot(out1, w2_ref[...], preferred_element_type=jnp.float32)
    q0 = q0_ref[...]
    sm = jnp.zeros((out1.shape[0], 64), jnp.float32)
    for hh in range(HEADS):
        blk = hw2[:, 64 * hh:64 * hh + 64]
        sm = sm + q0[0:1, hh:hh + 1] * blk
        ta_ref[:, 32 * hh:32 * hh + 32] = blk[:, 0:32]
        tb_ref[:, 32 * hh:32 * hh + 32] = blk[:, 32:64]
    sm2_ref[...] = sm
    hu = jnp.dot(out1, u2_ref[...], preferred_element_type=jnp.float32)
    hu_ref[...] = jnp.concatenate([hu, jnp.zeros_like(hu)], axis=1)


def _dense2(acc1, dega, sm1, b1, w2, u2, q0):
    return pl.pallas_call(
        _dense2_body,
        grid=(NBLK,),
        in_specs=[
            pl.BlockSpec((2, BLK, 32), lambda i: (0, i, 0)),
            pl.BlockSpec((2, BLK, 16), lambda i: (0, i, 0)),
            pl.BlockSpec((BLK, 32), lambda i: (i, 0)),
            pl.BlockSpec((1, 32), lambda i: (0, 0)),
            pl.BlockSpec((32, 512), lambda i: (0, 0)),
            pl.BlockSpec((32, 8), lambda i: (0, 0)),
            pl.BlockSpec((1, 8), lambda i: (0, 0)),
        ],
        out_specs=[
            pl.BlockSpec((BLK, 256), lambda i: (i, 0)),
            pl.BlockSpec((BLK, 256), lambda i: (i, 0)),
            pl.BlockSpec((BLK, 16), lambda i: (i, 0)),
            pl.BlockSpec((BLK, 64), lambda i: (i, 0)),
            pl.BlockSpec((BLK, 1), lambda i: (i, 0)),
        ],
        out_shape=[
            jax.ShapeDtypeStruct((N, 256), jnp.float32),
            jax.ShapeDtypeStruct((N, 256), jnp.float32),
            jax.ShapeDtypeStruct((N, 16), jnp.float32),
            jax.ShapeDtypeStruct((N, 64), jnp.float32),
            jax.ShapeDtypeStruct((N, 1), jnp.float32),
        ],
    )(acc1, dega, sm1, b1, w2, u2, q0)


def _dense3_body(aa_ref, ab_ref, sm2_ref, deg_ref, b2_ref, oh_ref,
                 w3_ref, b3_ref, out_ref, sums_ref, cnt_ref):
    i = pl.program_id(0)

    @pl.when(i == 0)
    def _():
        sums_ref[...] = jnp.zeros_like(sums_ref)
        cnt_ref[...] = jnp.zeros_like(cnt_ref)

    deg = deg_ref[...]
    a = (aa_ref[0] + aa_ref[1] + sm2_ref[:, 0:32]) / deg
    b = (ab_ref[0] + ab_ref[1] + sm2_ref[:, 32:64]) / deg
    out2 = jnp.maximum(jnp.concatenate([a, b], axis=1) + b2_ref[...], 0.0)
    oh = oh_ref[...]
    sums_ref[...] += lax.dot_general(oh, out2, (((0,), (0,)), ((), ())),
                                     preferred_element_type=jnp.float32)
    cnt_ref[...] += lax.dot_general(oh, jnp.ones_like(oh),
                                    (((0,), (0,)), ((), ())),
                                    preferred_element_type=jnp.float32)

    @pl.when(i == NBLK - 1)
    def _():
        g = sums_ref[...] / jnp.maximum(cnt_ref[:, 0:1], 1.0)
        out_ref[...] = jnp.dot(g, w3_ref[...],
                               preferred_element_type=jnp.float32) + b3_ref[...]


def _dense3(accA, accB, sm2, deg, b2, oh, w3, b3):
    return pl.pallas_call(
        _dense3_body,
        grid=(NBLK,),
        in_specs=[
            pl.BlockSpec((2, BLK, 32), lambda i: (0, i, 0)),
            pl.BlockSpec((2, BLK, 32), lambda i: (0, i, 0)),
            pl.BlockSpec((BLK, 64), lambda i: (i, 0)),
            pl.BlockSpec((BLK, 1), lambda i: (i, 0)),
            pl.BlockSpec((1, 64), lambda i: (0, 0)),
            pl.BlockSpec((BLK, 16), lambda i: (i, 0)),
            pl.BlockSpec((64, 16), lambda i: (0, 0)),
            pl.BlockSpec((1, 16), lambda i: (0, 0)),
        ],
        out_specs=pl.BlockSpec((16, 16), lambda i: (0, 0)),
        out_shape=jax.ShapeDtypeStruct((16, 16), jnp.float32),
        scratch_shapes=[
            pltpu.VMEM((16, 64), jnp.float32),
            pltpu.VMEM((16, 16), jnp.float32),
        ],
    )(accA, accB, sm2, deg, b2, oh, w3, b3)


_SC_MESH = plsc.VectorSubcoreMesh(core_axis_name="c", subcore_axis_name="s")
_ROWS_PER_TILE = ACC_ROWS // SC_TILES
_TAKE_DN = lax.GatherDimensionNumbers(
    offset_dims=(), collapsed_slice_dims=(0,), start_index_map=(0,))


def _take(v, idx):
    return lax.gather(v, idx[:, None], _TAKE_DN, (1,),
                      mode=lax.GatherScatterMode.PROMISE_IN_BOUNDS)


@functools.partial(
    pl.kernel,
    out_type=jax.ShapeDtypeStruct((SC_CORES * ACC_ROWS, 32), jnp.float32),
    mesh=_SC_MESH,
    scratch_types=[
        pltpu.VMEM((2, 2, CHUNK), jnp.int32),
        pltpu.VMEM((2, CHUNK, 16), jnp.float32),
        pltpu.VMEM((2, CHUNK, 16), jnp.float32),
        pltpu.VMEM((2, CHUNK, 256), jnp.float32),
        pltpu.VMEM((2, CHUNK, 32), jnp.float32),
        pltpu.VMEM((16,), jnp.float32),
        pltpu.VMEM_SHARED((ACC_ROWS, 32), jnp.float32),
        pltpu.SemaphoreType.DMA,
        pltpu.SemaphoreType.DMA,
        pltpu.SemaphoreType.DMA,
        pltpu.SemaphoreType.DMA,
    ],
    compiler_params=pltpu.CompilerParams(use_tc_tiling_on_sc=False),
)
def _edge_sc(e2_hbm, hu_hbm, tab_hbm, cc_hbm, out_hbm,
             sd, hus, hud, rows, msg, ccv, acc, semi0, semi1, semg0, semg1):
    cid = lax.axis_index("c")
    sid = lax.axis_index("s")
    wid = sid * SC_CORES + cid
    iota = lax.iota(jnp.int32, 16)
    semi = (semi0, semi1)
    semg = (semg0, semg1)

    zv = jnp.zeros((16,), jnp.float32)

    def _zrow(i, _):
        msg[0, i, pl.ds(0, 16)] = zv
        msg[0, i, pl.ds(16, 16)] = zv
        return 0

    lax.fori_loop(0, CHUNK, _zrow, 0)

    def _zacc(z, _):
        pltpu.sync_copy(msg.at[0],
                        acc.at[pl.ds(sid * _ROWS_PER_TILE + z * CHUNK,
                                     CHUNK)])
        return 0

    lax.fori_loop(0, _ROWS_PER_TILE // CHUNK, _zacc, 0)

    pltpu.sync_copy(cc_hbm, ccv)
    cc = ccv[...]
    r8 = (iota + 8) & 15
    r4 = (iota + 4) & 15
    r2 = (iota + 2) & 15
    r1 = (iota + 1) & 15
    hsplat = [jnp.full((16,), hh, jnp.int32) for hh in range(HEADS)]
    plsc.subcore_barrier()

    tile_base = wid * (NCHUNK * CHUNK)

    def _issue_idx(c, b):
        pltpu.async_copy(
            e2_hbm.at[:, pl.ds(tile_base + c * CHUNK, CHUNK)], sd.at[b],
            semi[b])

    def _wait_idx(b):
        pltpu.make_async_copy(
            e2_hbm.at[:, pl.ds(tile_base, CHUNK)], sd.at[b], semi[b]).wait()

    def _issue_gathers(b):
        pltpu.async_copy(hu_hbm.at[sd.at[b, 0]], hus.at[b], semg[b])
        pltpu.async_copy(hu_hbm.at[sd.at[b, 1]], hud.at[b], semg[b])
        pltpu.async_copy(tab_hbm.at[sd.at[b, 0]], rows.at[b], semg[b])

    def _wait_gathers(b):
        pltpu.make_async_copy(hu_hbm.at[sd.at[b, 0]], hus.at[b],
                              semg[b]).wait()
        pltpu.make_async_copy(hu_hbm.at[sd.at[b, 1]], hud.at[b],
                              semg[b]).wait()
        pltpu.make_async_copy(tab_hbm.at[sd.at[b, 0]], rows.at[b],
                              semg[b]).wait()

    def _compute(b):
        def _pair(i, _):
            for e in (2 * i, 2 * i + 1):
                t = jnp.exp(hus[b, e] - hud[b, e] + cc)
                s = t + _take(t, r8)
                s = s + _take(s, r4)
                s = s + _take(s, r2)
                s = s + _take(s, r1)
                th = _take(t, hsplat[0])
                m0 = th * rows[b, e, pl.ds(0, 16)]
                m1 = th * rows[b, e, pl.ds(16, 16)]
                for hh in range(1, HEADS):
                    th = _take(t, hsplat[hh])
                    m0 = m0 + th * rows[b, e, pl.ds(32 * hh, 16)]
                    m1 = m1 + th * rows[b, e, pl.ds(32 * hh + 16, 16)]
                msg[b, e, pl.ds(0, 16)] = m0 / s
                msg[b, e, pl.ds(16, 16)] = m1 / s
            return 0

        lax.fori_loop(0, CHUNK // 2, _pair, 0)

    def _scatter(b):
        pltpu.sync_copy(msg.at[b], acc.at[sd.at[b, 1]], add=True)

    _issue_idx(0, 0)
    _issue_idx(1, 1)
    _wait_idx(0)
    _issue_gathers(0)

    def _body(c, b):
        bn = 1 - b

        @pl.when(c + 1 < NCHUNK)
        def _():
            _wait_idx(bn)
            _issue_gathers(bn)

        _wait_gathers(b)
        _compute(b)
        _scatter(b)

        @pl.when(c + 2 < NCHUNK)
        def _():
            _issue_idx(c + 2, b)

    def _pair_body(g, _):
        _body(2 * g, 0)
        _body(2 * g + 1, 1)
        return 0

    lax.fori_loop(0, NCHUNK // 2, _pair_body, 0)
    plsc.subcore_barrier()

    def _copyout(z, _):
        r0 = sid * _ROWS_PER_TILE + z * ZCH
        pltpu.sync_copy(acc.at[pl.ds(r0, ZCH)],
                        out_hbm.at[pl.ds(cid * ACC_ROWS + r0, ZCH)])
        return 0

    lax.fori_loop(0, ZN, _copyout, 0)


@functools.partial(
    pl.kernel,
    out_type=jax.ShapeDtypeStruct((SC_CORES * ACC_ROWS, 16), jnp.float32),
    mesh=_SC_MESH,
    scratch_types=[
        pltpu.VMEM((2, DCH), jnp.int32),
        pltpu.VMEM((DCH, 16), jnp.float32),
        pltpu.VMEM((ZCH, 16), jnp.float32),
        pltpu.VMEM_SHARED((ACC_ROWS, 16), jnp.float32),
        pltpu.SemaphoreType.DMA,
        pltpu.SemaphoreType.DMA,
    ],
    compiler_params=pltpu.CompilerParams(use_tc_tiling_on_sc=False),
)
def _deg_sc(dst_hbm, out_hbm, didx, ones, zrows, acc, semd0, semd1):
    cid = lax.axis_index("c")
    sid = lax.axis_index("s")
    wid = sid * SC_CORES + cid
    iota = lax.iota(jnp.int32, 16)
    onev = jnp.where(iota == 0, 1.0, 0.0).astype(jnp.float32)
    zv = jnp.zeros((16,), jnp.float32)
    semd = (semd0, semd1)

    def _fill(i, _):
        ones[i, pl.ds(0, 16)] = onev
        return 0

    lax.fori_loop(0, DCH, _fill, 0)

    def _fillz(i, _):
        zrows[i, pl.ds(0, 16)] = zv
        return 0

    lax.fori_loop(0, ZCH, _fillz, 0)

    def _zacc(z, _):
        pltpu.sync_copy(zrows, acc.at[pl.ds(sid * _ROWS_PER_TILE + z * ZCH,
                                            ZCH)])
        return 0

    lax.fori_loop(0, ZN, _zacc, 0)
    plsc.subcore_barrier()

    tile_base = wid * (DNCH * DCH)

    def _dissue(c, b):
        pltpu.async_copy(dst_hbm.at[pl.ds(tile_base + c * DCH, DCH)],
                         didx.at[b], semd[b])

    def _dwait(b):
        pltpu.make_async_copy(dst_hbm.at[pl.ds(tile_base, DCH)], didx.at[b],
                              semd[b]).wait()

    _dissue(0, 0)
    _dissue(1, 1)

    def _dbody(c, b):
        _dwait(b)
        pltpu.sync_copy(ones, acc.at[didx.at[b]], add=True)

        @pl.when(c + 2 < DNCH)
        def _():
            _dissue(c + 2, b)

    def _dpair(g, _):
        _dbody(2 * g, 0)
        _dbody(2 * g + 1, 1)
        return 0

    lax.fori_loop(0, DNCH // 2, _dpair, 0)
    plsc.subcore_barrier()

    def _copyout(z, _):
        r0 = sid * _ROWS_PER_TILE + z * ZCH
        pltpu.sync_copy(acc.at[pl.ds(r0, ZCH)],
                        out_hbm.at[pl.ds(cid * ACC_ROWS + r0, ZCH)])
        return 0

    lax.fori_loop(0, ZN, _copyout, 0)


def _edge_jnp(src, dst, hu_pad, table, cc, msg_w):
    hs = hu_pad[src][:, 0:8]
    hd = hu_pad[dst][:, 0:8]
    q = jax.nn.softmax(hs - hd + cc[None, 0:8], axis=-1)
    rows = table[src]
    msg = jnp.zeros((src.shape[0], 32), jnp.float32)
    for hh in range(HEADS):
        msg = msg + q[:, hh:hh + 1] * rows[:, 32 * hh:32 * hh + 32]
    if msg_w == 40:
        msg = jnp.concatenate(
            [msg, jnp.ones((src.shape[0], 1), jnp.float32),
             jnp.zeros((src.shape[0], 7), jnp.float32)], axis=1)
    acc = jax.ops.segment_sum(msg, dst, num_segments=ACC_ROWS)
    return jnp.stack([acc, jnp.zeros_like(acc)])


def kernel(x, edge_index, batch, fc0_w, fc0_b, conv1_weight, conv1_u, conv1_c,
           conv1_bias, conv2_weight, conv2_u, conv2_c, conv2_bias, fc1_w,
           fc1_b):
    f32 = jnp.float32
    q01 = jax.nn.softmax(conv1_c)[None, :]
    q02 = jax.nn.softmax(conv2_c)[None, :]
    cc1 = jnp.concatenate([conv1_c, jnp.full((8,), -1e30, f32)])
    cc2 = jnp.concatenate([conv2_c, jnp.full((8,), -1e30, f32)])

    src = jnp.concatenate(
        [edge_index[0], jnp.zeros((EPAD - E,), jnp.int32)])
    dst = jnp.concatenate(
        [edge_index[1], jnp.full((EPAD - E,), DUMMY_DST, jnp.int32)])
    e2 = jnp.stack([src, dst])

    hw1, hu1, sm1 = _dense1(x, fc0_w, fc0_b[None, :], conv1_weight, conv1_u,
                            q01)
    hu1p = jnp.zeros((HU_ROWS, 16), f32).at[0:N].set(hu1)
    dega = _deg_sc(dst).reshape(2, ACC_ROWS, 16)
    acc1 = _edge_sc(e2, hu1p, hw1, cc1).reshape(2, ACC_ROWS, 32)

    ta, tb, hu2, sm2, deg = _dense2(acc1, dega, sm1, conv1_bias[None, :],
                                    conv2_weight, conv2_u, q02)
    hu2p = jnp.zeros((HU_ROWS, 16), f32).at[0:N].set(hu2)
    accA = _edge_sc(e2, hu2p, ta, cc2).reshape(2, ACC_ROWS, 32)
    accB = _edge_sc(e2, hu2p, tb, cc2).reshape(2, ACC_ROWS, 32)

    oh = (batch[:, None] == jnp.arange(16, dtype=jnp.int32)[None, :]
          ).astype(f32)
    out16 = _dense3(accA, accB, sm2, deg, conv2_bias[None, :], oh,
                    jnp.zeros((64, 16), f32).at[:, 0:10].set(fc1_w),
                    jnp.zeros((1, 16), f32).at[0, 0:10].set(fc1_b))
    return out16[:, 0:10]


# speedup vs baseline: 4.3358x; 1.0474x over previous
import functools

import jax
import jax.numpy as jnp
from jax import lax
from jax.experimental import pallas as pl
from jax.experimental.pallas import tpu as pltpu
from jax.experimental.pallas import tpu_sc as plsc

N = 50000
E = 800000
HEADS = 8
BLK = 1000
NBLK = N // BLK

SC_CORES = 2
SC_TILES = 16
SC_WORKERS = SC_CORES * SC_TILES
CHUNK = 32
NCHUNK = 784
EPAD = SC_WORKERS * CHUNK * NCHUNK
DCH = 128
DNCH = EPAD // (SC_WORKERS * DCH)
ACC_ROWS = 50176
ZCH = 64
ZN = ACC_ROWS // SC_TILES // ZCH
HU_ROWS = 50016
DUMMY_DST = N + 8


def _dense1_body(x_ref, w0_ref, b0_ref, w1_ref, u1_ref, q0_ref,
                 hw_ref, hu_ref, sm_ref):
    h = jnp.maximum(jnp.dot(x_ref[...], w0_ref[...],
                            preferred_element_type=jnp.float32)
                    + b0_ref[...], 0.0)
    hw = jnp.dot(h, w1_ref[...], preferred_element_type=jnp.float32)
    hw_ref[...] = hw
    hu = jnp.dot(h, u1_ref[...], preferred_element_type=jnp.float32)
    hu_ref[...] = jnp.concatenate([hu, jnp.zeros_like(hu)], axis=1)
    q0 = q0_ref[...]
    sm = jnp.zeros((x_ref.shape[0], 32), jnp.float32)
    for hh in range(HEADS):
        sm = sm + q0[0:1, hh:hh + 1] * hw[:, 32 * hh:32 * hh + 32]
    sm_ref[...] = sm


def _dense1(x, w0, b0, w1, u1, q0):
    return pl.pallas_call(
        _dense1_body,
        grid=(NBLK,),
        in_specs=[
            pl.BlockSpec((BLK, 128), lambda i: (i, 0)),
            pl.BlockSpec((128, 16), lambda i: (0, 0)),
            pl.BlockSpec((1, 16), lambda i: (0, 0)),
            pl.BlockSpec((16, 256), lambda i: (0, 0)),
            pl.BlockSpec((16, 8), lambda i: (0, 0)),
            pl.BlockSpec((1, 8), lambda i: (0, 0)),
        ],
        out_specs=[
            pl.BlockSpec((BLK, 256), lambda i: (i, 0)),
            pl.BlockSpec((BLK, 16), lambda i: (i, 0)),
            pl.BlockSpec((BLK, 32), lambda i: (i, 0)),
        ],
        out_shape=[
            jax.ShapeDtypeStruct((N, 256), jnp.float32),
            jax.ShapeDtypeStruct((N, 16), jnp.float32),
            jax.ShapeDtypeStruct((N, 32), jnp.float32),
        ],
    )(x, w0, b0, w1, u1, q0)


def _dense2_body(acc_ref, dg_ref, sm1_ref, b1_ref, w2_ref, u2_ref, q0_ref,
                 ta_ref, tb_ref, hu_ref, sm2_ref, deg_ref):
    a = acc_ref[0] + acc_ref[1] + sm1_ref[...]
    deg = dg_ref[0, :, 0:1] + dg_ref[1, :, 0:1] + 1.0
    deg_ref[...] = deg
    out1 = jnp.maximum(a / deg + b1_ref[...], 0.0)
    hw2 = jnp.dot(out1, w2_ref[...], preferred_element_type=jnp.float32)
    q0 = q0_ref[...]
    sm = jnp.zeros((out1.shape[0], 64), jnp.float32)
    for hh in range(HEADS):
        blk = hw2[:, 64 * hh:64 * hh + 64]
        sm = sm + q0[0:1, hh:hh + 1] * blk
        ta_ref[:, 32 * hh:32 * hh + 32] = blk[:, 0:32]
        tb_ref[:, 32 * hh:32 * hh + 32] = blk[:, 32:64]
    sm2_ref[...] = sm
    hu = jnp.dot(out1, u2_ref[...], preferred_element_type=jnp.float32)
    hu_ref[...] = jnp.concatenate([hu, jnp.zeros_like(hu)], axis=1)


def _dense2(acc1, dega, sm1, b1, w2, u2, q0):
    return pl.pallas_call(
        _dense2_body,
        grid=(NBLK,),
        in_specs=[
            pl.BlockSpec((2, BLK, 32), lambda i: (0, i, 0)),
            pl.BlockSpec((2, BLK, 16), lambda i: (0, i, 0)),
            pl.BlockSpec((BLK, 32), lambda i: (i, 0)),
            pl.BlockSpec((1, 32), lambda i: (0, 0)),
            pl.BlockSpec((32, 512), lambda i: (0, 0)),
            pl.BlockSpec((32, 8), lambda i: (0, 0)),
            pl.BlockSpec((1, 8), lambda i: (0, 0)),
        ],
        out_specs=[
            pl.BlockSpec((BLK, 256), lambda i: (i, 0)),
            pl.BlockSpec((BLK, 256), lambda i: (i, 0)),
            pl.BlockSpec((BLK, 16), lambda i: (i, 0)),
            pl.BlockSpec((BLK, 64), lambda i: (i, 0)),
            pl.BlockSpec((BLK, 1), lambda i: (i, 0)),
        ],
        out_shape=[
            jax.ShapeDtypeStruct((N, 256), jnp.float32),
            jax.ShapeDtypeStruct((N, 256), jnp.float32),
            jax.ShapeDtypeStruct((N, 16), jnp.float32),
            jax.ShapeDtypeStruct((N, 64), jnp.float32),
            jax.ShapeDtypeStruct((N, 1), jnp.float32),
        ],
    )(acc1, dega, sm1, b1, w2, u2, q0)


def _dense3_body(aa_ref, ab_ref, sm2_ref, deg_ref, b2_ref, oh_ref,
                 w3_ref, b3_ref, out_ref, sums_ref, cnt_ref):
    i = pl.program_id(0)

    @pl.when(i == 0)
    def _():
        sums_ref[...] = jnp.zeros_like(sums_ref)
        cnt_ref[...] = jnp.zeros_like(cnt_ref)

    deg = deg_ref[...]
    a = (aa_ref[0] + aa_ref[1] + sm2_ref[:, 0:32]) / deg
    b = (ab_ref[0] + ab_ref[1] + sm2_ref[:, 32:64]) / deg
    out2 = jnp.maximum(jnp.concatenate([a, b], axis=1) + b2_ref[...], 0.0)
    oh = oh_ref[...]
    sums_ref[...] += lax.dot_general(oh, out2, (((0,), (0,)), ((), ())),
                                     preferred_element_type=jnp.float32)
    cnt_ref[...] += lax.dot_general(oh, jnp.ones_like(oh),
                                    (((0,), (0,)), ((), ())),
                                    preferred_element_type=jnp.float32)

    @pl.when(i == NBLK - 1)
    def _():
        g = sums_ref[...] / jnp.maximum(cnt_ref[:, 0:1], 1.0)
        out_ref[...] = jnp.dot(g, w3_ref[...],
                               preferred_element_type=jnp.float32) + b3_ref[...]


def _dense3(accA, accB, sm2, deg, b2, oh, w3, b3):
    return pl.pallas_call(
        _dense3_body,
        grid=(NBLK,),
        in_specs=[
            pl.BlockSpec((2, BLK, 32), lambda i: (0, i, 0)),
            pl.BlockSpec((2, BLK, 32), lambda i: (0, i, 0)),
            pl.BlockSpec((BLK, 64), lambda i: (i, 0)),
            pl.BlockSpec((BLK, 1), lambda i: (i, 0)),
            pl.BlockSpec((1, 64), lambda i: (0, 0)),
            pl.BlockSpec((BLK, 16), lambda i: (i, 0)),
            pl.BlockSpec((64, 16), lambda i: (0, 0)),
            pl.BlockSpec((1, 16), lambda i: (0, 0)),
        ],
        out_specs=pl.BlockSpec((16, 16), lambda i: (0, 0)),
        out_shape=jax.ShapeDtypeStruct((16, 16), jnp.float32),
        scratch_shapes=[
            pltpu.VMEM((16, 64), jnp.float32),
            pltpu.VMEM((16, 16), jnp.float32),
        ],
    )(accA, accB, sm2, deg, b2, oh, w3, b3)


_SC_MESH = plsc.VectorSubcoreMesh(core_axis_name="c", subcore_axis_name="s")
_ROWS_PER_TILE = ACC_ROWS // SC_TILES
_TAKE_DN = lax.GatherDimensionNumbers(
    offset_dims=(), collapsed_slice_dims=(0,), start_index_map=(0,))


def _take(v, idx):
    return lax.gather(v, idx[:, None], _TAKE_DN, (1,),
                      mode=lax.GatherScatterMode.PROMISE_IN_BOUNDS)


@functools.partial(
    pl.kernel,
    out_type=jax.ShapeDtypeStruct((SC_CORES * ACC_ROWS, 32), jnp.float32),
    mesh=_SC_MESH,
    scratch_types=[
        pltpu.VMEM((2, 2, CHUNK), jnp.int32),
        pltpu.VMEM((2, CHUNK, 16), jnp.float32),
        pltpu.VMEM((2, CHUNK, 16), jnp.float32),
        pltpu.VMEM((2, CHUNK, 256), jnp.float32),
        pltpu.VMEM((2, CHUNK, 32), jnp.float32),
        pltpu.VMEM((16,), jnp.float32),
        pltpu.VMEM_SHARED((ACC_ROWS, 32), jnp.float32),
        pltpu.SemaphoreType.DMA,
        pltpu.SemaphoreType.DMA,
        pltpu.SemaphoreType.DMA,
        pltpu.SemaphoreType.DMA,
    ],
    compiler_params=pltpu.CompilerParams(use_tc_tiling_on_sc=False),
)
def _edge_sc(e2_hbm, hu_hbm, tab_hbm, cc_hbm, out_hbm,
             sd, hus, hud, rows, msg, ccv, acc, semi0, semi1, semg0, semg1):
    cid = lax.axis_index("c")
    sid = lax.axis_index("s")
    wid = sid * SC_CORES + cid
    iota = lax.iota(jnp.int32, 16)
    semi = (semi0, semi1)
    semg = (semg0, semg1)

    zv = jnp.zeros((16,), jnp.float32)

    def _zrow(i, _):
        msg[0, i, pl.ds(0, 16)] = zv
        msg[0, i, pl.ds(16, 16)] = zv
        return 0

    lax.fori_loop(0, CHUNK, _zrow, 0)

    def _zacc(z, _):
        pltpu.sync_copy(msg.at[0],
                        acc.at[pl.ds(sid * _ROWS_PER_TILE + z * CHUNK,
                                     CHUNK)])
        return 0

    lax.fori_loop(0, _ROWS_PER_TILE // CHUNK, _zacc, 0)

    pltpu.sync_copy(cc_hbm, ccv)
    cc = ccv[...]
    hsplat = [jnp.full((16,), hh, jnp.int32) for hh in range(HEADS)]
    plsc.subcore_barrier()

    tile_base = wid * (NCHUNK * CHUNK)

    def _issue_idx(c, b):
        pltpu.async_copy(
            e2_hbm.at[:, pl.ds(tile_base + c * CHUNK, CHUNK)], sd.at[b],
            semi[b])

    def _wait_idx(b):
        pltpu.make_async_copy(
            e2_hbm.at[:, pl.ds(tile_base, CHUNK)], sd.at[b], semi[b]).wait()

    def _issue_gathers(b):
        pltpu.async_copy(hu_hbm.at[sd.at[b, 0]], hus.at[b], semg[b])
        pltpu.async_copy(hu_hbm.at[sd.at[b, 1]], hud.at[b], semg[b])
        pltpu.async_copy(tab_hbm.at[sd.at[b, 0]], rows.at[b], semg[b])

    def _wait_gathers(b):
        pltpu.make_async_copy(hu_hbm.at[sd.at[b, 0]], hus.at[b],
                              semg[b]).wait()
        pltpu.make_async_copy(hu_hbm.at[sd.at[b, 1]], hud.at[b],
                              semg[b]).wait()
        pltpu.make_async_copy(tab_hbm.at[sd.at[b, 0]], rows.at[b],
                              semg[b]).wait()

    def _compute(b):
        def _quad(i, _):
            for e in (4 * i, 4 * i + 1, 4 * i + 2, 4 * i + 3):
                t = jnp.exp(hus[b, e] - hud[b, e] + cc)
                th = [_take(t, hsplat[hh]) for hh in range(HEADS)]
                s = ((th[0] + th[1]) + (th[2] + th[3])) + (
                    (th[4] + th[5]) + (th[6] + th[7]))
                p0 = [th[hh] * rows[b, e, pl.ds(32 * hh, 16)]
                      for hh in range(HEADS)]
                p1 = [th[hh] * rows[b, e, pl.ds(32 * hh + 16, 16)]
                      for hh in range(HEADS)]
                m0 = ((p0[0] + p0[1]) + (p0[2] + p0[3])) + (
                    (p0[4] + p0[5]) + (p0[6] + p0[7]))
                m1 = ((p1[0] + p1[1]) + (p1[2] + p1[3])) + (
                    (p1[4] + p1[5]) + (p1[6] + p1[7]))
                msg[b, e, pl.ds(0, 16)] = m0 / s
                msg[b, e, pl.ds(16, 16)] = m1 / s
            return 0

        lax.fori_loop(0, CHUNK // 4, _quad, 0)

    def _scatter(b):
        pltpu.sync_copy(msg.at[b], acc.at[sd.at[b, 1]], add=True)

    _issue_idx(0, 0)
    _issue_idx(1, 1)
    _wait_idx(0)
    _issue_gathers(0)

    def _body(c, b):
        bn = 1 - b

        @pl.when(c + 1 < NCHUNK)
        def _():
            _wait_idx(bn)
            _issue_gathers(bn)

        _wait_gathers(b)
        _compute(b)
        _scatter(b)

        @pl.when(c + 2 < NCHUNK)
        def _():
            _issue_idx(c + 2, b)

    def _pair_body(g, _):
        _body(2 * g, 0)
        _body(2 * g + 1, 1)
        return 0

    lax.fori_loop(0, NCHUNK // 2, _pair_body, 0)
    plsc.subcore_barrier()

    def _copyout(z, _):
        r0 = sid * _ROWS_PER_TILE + z * ZCH
        pltpu.sync_copy(acc.at[pl.ds(r0, ZCH)],
                        out_hbm.at[pl.ds(cid * ACC_ROWS + r0, ZCH)])
        return 0

    lax.fori_loop(0, ZN, _copyout, 0)


@functools.partial(
    pl.kernel,
    out_type=jax.ShapeDtypeStruct((SC_CORES * ACC_ROWS, 16), jnp.float32),
    mesh=_SC_MESH,
    scratch_types=[
        pltpu.VMEM((2, DCH), jnp.int32),
        pltpu.VMEM((DCH, 16), jnp.float32),
        pltpu.VMEM((ZCH, 16), jnp.float32),
        pltpu.VMEM_SHARED((ACC_ROWS, 16), jnp.float32),
        pltpu.SemaphoreType.DMA,
        pltpu.SemaphoreType.DMA,
    ],
    compiler_params=pltpu.CompilerParams(use_tc_tiling_on_sc=False),
)
def _deg_sc(dst_hbm, out_hbm, didx, ones, zrows, acc, semd0, semd1):
    cid = lax.axis_index("c")
    sid = lax.axis_index("s")
    wid = sid * SC_CORES + cid
    iota = lax.iota(jnp.int32, 16)
    onev = jnp.where(iota == 0, 1.0, 0.0).astype(jnp.float32)
    zv = jnp.zeros((16,), jnp.float32)
    semd = (semd0, semd1)

    def _fill(i, _):
        ones[i, pl.ds(0, 16)] = onev
        return 0

    lax.fori_loop(0, DCH, _fill, 0)

    def _fillz(i, _):
        zrows[i, pl.ds(0, 16)] = zv
        return 0

    lax.fori_loop(0, ZCH, _fillz, 0)

    def _zacc(z, _):
        pltpu.sync_copy(zrows, acc.at[pl.ds(sid * _ROWS_PER_TILE + z * ZCH,
                                            ZCH)])
        return 0

    lax.fori_loop(0, ZN, _zacc, 0)
    plsc.subcore_barrier()

    tile_base = wid * (DNCH * DCH)

    def _dissue(c, b):
        pltpu.async_copy(dst_hbm.at[pl.ds(tile_base + c * DCH, DCH)],
                         didx.at[b], semd[b])

    def _dwait(b):
        pltpu.make_async_copy(dst_hbm.at[pl.ds(tile_base, DCH)], didx.at[b],
                              semd[b]).wait()

    _dissue(0, 0)
    _dissue(1, 1)

    def _dbody(c, b):
        _dwait(b)
        pltpu.sync_copy(ones, acc.at[didx.at[b]], add=True)

        @pl.when(c + 2 < DNCH)
        def _():
            _dissue(c + 2, b)

    def _dpair(g, _):
        _dbody(2 * g, 0)
        _dbody(2 * g + 1, 1)
        return 0

    lax.fori_loop(0, DNCH // 2, _dpair, 0)
    plsc.subcore_barrier()

    def _copyout(z, _):
        r0 = sid * _ROWS_PER_TILE + z * ZCH
        pltpu.sync_copy(acc.at[pl.ds(r0, ZCH)],
                        out_hbm.at[pl.ds(cid * ACC_ROWS + r0, ZCH)])
        return 0

    lax.fori_loop(0, ZN, _copyout, 0)


def _edge_jnp(src, dst, hu_pad, table, cc, msg_w):
    hs = hu_pad[src][:, 0:8]
    hd = hu_pad[dst][:, 0:8]
    q = jax.nn.softmax(hs - hd + cc[None, 0:8], axis=-1)
    rows = table[src]
    msg = jnp.zeros((src.shape[0], 32), jnp.float32)
    for hh in range(HEADS):
        msg = msg + q[:, hh:hh + 1] * rows[:, 32 * hh:32 * hh + 32]
    if msg_w == 40:
        msg = jnp.concatenate(
            [msg, jnp.ones((src.shape[0], 1), jnp.float32),
             jnp.zeros((src.shape[0], 7), jnp.float32)], axis=1)
    acc = jax.ops.segment_sum(msg, dst, num_segments=ACC_ROWS)
    return jnp.stack([acc, jnp.zeros_like(acc)])


def kernel(x, edge_index, batch, fc0_w, fc0_b, conv1_weight, conv1_u, conv1_c,
           conv1_bias, conv2_weight, conv2_u, conv2_c, conv2_bias, fc1_w,
           fc1_b):
    f32 = jnp.float32
    q01 = jax.nn.softmax(conv1_c)[None, :]
    q02 = jax.nn.softmax(conv2_c)[None, :]
    cc1 = jnp.concatenate([conv1_c, jnp.full((8,), -1e30, f32)])
    cc2 = jnp.concatenate([conv2_c, jnp.full((8,), -1e30, f32)])

    src = jnp.concatenate(
        [edge_index[0], jnp.zeros((EPAD - E,), jnp.int32)])
    dst = jnp.concatenate(
        [edge_index[1], jnp.full((EPAD - E,), DUMMY_DST, jnp.int32)])
    e2 = jnp.stack([src, dst])

    hw1, hu1, sm1 = _dense1(x, fc0_w, fc0_b[None, :], conv1_weight, conv1_u,
                            q01)
    hu1p = jnp.zeros((HU_ROWS, 16), f32).at[0:N].set(hu1)
    dega = _deg_sc(dst).reshape(2, ACC_ROWS, 16)
    acc1 = _edge_sc(e2, hu1p, hw1, cc1).reshape(2, ACC_ROWS, 32)

    ta, tb, hu2, sm2, deg = _dense2(acc1, dega, sm1, conv1_bias[None, :],
                                    conv2_weight, conv2_u, q02)
    hu2p = jnp.zeros((HU_ROWS, 16), f32).at[0:N].set(hu2)
    accA = _edge_sc(e2, hu2p, ta, cc2).reshape(2, ACC_ROWS, 32)
    accB = _edge_sc(e2, hu2p, tb, cc2).reshape(2, ACC_ROWS, 32)

    oh = (batch[:, None] == jnp.arange(16, dtype=jnp.int32)[None, :]
          ).astype(f32)
    out16 = _dense3(accA, accB, sm2, deg, conv2_bias[None, :], oh,
                    jnp.zeros((64, 16), f32).at[:, 0:10].set(fc1_w),
                    jnp.zeros((1, 16), f32).at[0, 0:10].set(fc1_b))
    return out16[:, 0:10]


# speedup vs baseline: 4.8563x; 1.1200x over previous
import functools

import jax
import jax.numpy as jnp
import numpy as np
from jax import lax
from jax.experimental import pallas as pl
from jax.experimental.pallas import tpu as pltpu
from jax.experimental.pallas import tpu_sc as plsc

N = 50000
E = 800000
HEADS = 8
BLK = 2000
NBLK = N // BLK

SC_CORES = 2
SC_TILES = 16
SC_WORKERS = SC_CORES * SC_TILES
CHUNK = 64
NCHUNK = 392
EPAD = SC_WORKERS * CHUNK * NCHUNK
DCH = 128
DNCH = EPAD // (SC_WORKERS * DCH)
ACC_ROWS = 50176
ZCH = 64
ZN = ACC_ROWS // SC_TILES // ZCH
HU_ROWS = 50016
DUMMY_DST = N + 8


def _dense1_body(x_ref, w0_ref, b0_ref, w1_ref, u1_ref, q0_ref,
                 hw_ref, hu_ref, sm_ref):
    h = jnp.maximum(jnp.dot(x_ref[...], w0_ref[...],
                            preferred_element_type=jnp.float32)
                    + b0_ref[...], 0.0)
    hw = jnp.dot(h, w1_ref[...], preferred_element_type=jnp.float32)
    hw_ref[...] = hw
    hu = jnp.dot(h, u1_ref[...], preferred_element_type=jnp.float32)
    hu_ref[...] = jnp.concatenate([hu, jnp.zeros_like(hu)], axis=1)
    q0 = q0_ref[...]
    sm = jnp.zeros((x_ref.shape[0], 32), jnp.float32)
    for hh in range(HEADS):
        sm = sm + q0[0:1, hh:hh + 1] * hw[:, 32 * hh:32 * hh + 32]
    sm_ref[...] = sm


def _dense1(x, w0, b0, w1, u1, q0):
    return pl.pallas_call(
        _dense1_body,
        grid=(NBLK,),
        in_specs=[
            pl.BlockSpec((BLK, 128), lambda i: (i, 0)),
            pl.BlockSpec((128, 16), lambda i: (0, 0)),
            pl.BlockSpec((1, 16), lambda i: (0, 0)),
            pl.BlockSpec((16, 256), lambda i: (0, 0)),
            pl.BlockSpec((16, 8), lambda i: (0, 0)),
            pl.BlockSpec((1, 8), lambda i: (0, 0)),
        ],
        out_specs=[
            pl.BlockSpec((BLK, 256), lambda i: (i, 0)),
            pl.BlockSpec((BLK, 16), lambda i: (i, 0)),
            pl.BlockSpec((BLK, 32), lambda i: (i, 0)),
        ],
        out_shape=[
            jax.ShapeDtypeStruct((N, 256), jnp.float32),
            jax.ShapeDtypeStruct((N, 16), jnp.float32),
            jax.ShapeDtypeStruct((N, 32), jnp.float32),
        ],
    )(x, w0, b0, w1, u1, q0)


def _dense2_body(acc_ref, dg_ref, sm1_ref, b1_ref, w2_ref, u2_ref, q0_ref,
                 p64_ref, ta_ref, tb_ref, hu_ref, sm2_ref, deg_ref):
    a = acc_ref[0] + acc_ref[1] + sm1_ref[...]
    deg = dg_ref[0, :, 0:1] + dg_ref[1, :, 0:1] + 1.0
    deg_ref[...] = deg
    out1 = jnp.maximum(a / deg + b1_ref[...], 0.0)
    hw2 = jnp.dot(out1, w2_ref[...], preferred_element_type=jnp.float32)
    q0 = q0_ref[...]
    sm = jnp.zeros((out1.shape[0], 64), jnp.float32)
    for hh in range(HEADS):
        blk = hw2[:, 64 * hh:64 * hh + 64]
        sm = sm + q0[0:1, hh:hh + 1] * blk
        ta_ref[:, 32 * hh:32 * hh + 32] = blk[:, 0:32].astype(jnp.bfloat16)
        tb_ref[:, 32 * hh:32 * hh + 32] = blk[:, 32:64].astype(jnp.bfloat16)
    sm2_ref[...] = jnp.dot(sm, p64_ref[...],
                           preferred_element_type=jnp.float32)
    hu = jnp.dot(out1, u2_ref[...], preferred_element_type=jnp.float32)
    hu_ref[...] = jnp.concatenate([hu, jnp.zeros_like(hu)], axis=1)


def _dense2(acc1, dega, sm1, b1, w2, u2, q0, p64):
    return pl.pallas_call(
        _dense2_body,
        grid=(NBLK,),
        in_specs=[
            pl.BlockSpec((2, BLK, 32), lambda i: (0, i, 0)),
            pl.BlockSpec((2, BLK, 16), lambda i: (0, i, 0)),
            pl.BlockSpec((BLK, 32), lambda i: (i, 0)),
            pl.BlockSpec((1, 32), lambda i: (0, 0)),
            pl.BlockSpec((32, 512), lambda i: (0, 0)),
            pl.BlockSpec((32, 8), lambda i: (0, 0)),
            pl.BlockSpec((1, 8), lambda i: (0, 0)),
            pl.BlockSpec((64, 64), lambda i: (0, 0)),
        ],
        out_specs=[
            pl.BlockSpec((BLK, 256), lambda i: (i, 0)),
            pl.BlockSpec((BLK, 256), lambda i: (i, 0)),
            pl.BlockSpec((BLK, 16), lambda i: (i, 0)),
            pl.BlockSpec((BLK, 64), lambda i: (i, 0)),
            pl.BlockSpec((BLK, 1), lambda i: (i, 0)),
        ],
        out_shape=[
            jax.ShapeDtypeStruct((N, 256), jnp.bfloat16),
            jax.ShapeDtypeStruct((N, 256), jnp.bfloat16),
            jax.ShapeDtypeStruct((N, 16), jnp.float32),
            jax.ShapeDtypeStruct((N, 64), jnp.float32),
            jax.ShapeDtypeStruct((N, 1), jnp.float32),
        ],
    )(acc1, dega, sm1, b1, w2, u2, q0, p64)


def _dense3_body(aa_ref, ab_ref, sm2_ref, deg_ref, b2_ref, oh_ref,
                 w3_ref, b3_ref, out_ref, sums_ref, cnt_ref):
    i = pl.program_id(0)

    @pl.when(i == 0)
    def _():
        sums_ref[...] = jnp.zeros_like(sums_ref)
        cnt_ref[...] = jnp.zeros_like(cnt_ref)

    deg = deg_ref[...]
    a = (aa_ref[0] + aa_ref[1] + sm2_ref[:, 0:32]) / deg
    b = (ab_ref[0] + ab_ref[1] + sm2_ref[:, 32:64]) / deg
    out2 = jnp.maximum(jnp.concatenate([a, b], axis=1) + b2_ref[...], 0.0)
    oh = oh_ref[...]
    sums_ref[...] += lax.dot_general(oh, out2, (((0,), (0,)), ((), ())),
                                     preferred_element_type=jnp.float32)
    cnt_ref[...] += lax.dot_general(oh, jnp.ones_like(oh),
                                    (((0,), (0,)), ((), ())),
                                    preferred_element_type=jnp.float32)

    @pl.when(i == NBLK - 1)
    def _():
        g = sums_ref[...] / jnp.maximum(cnt_ref[:, 0:1], 1.0)
        out_ref[...] = jnp.dot(g, w3_ref[...],
                               preferred_element_type=jnp.float32) + b3_ref[...]


def _dense3(accA, accB, sm2, deg, b2, oh, w3, b3):
    return pl.pallas_call(
        _dense3_body,
        grid=(NBLK,),
        in_specs=[
            pl.BlockSpec((2, BLK, 32), lambda i: (0, i, 0)),
            pl.BlockSpec((2, BLK, 32), lambda i: (0, i, 0)),
            pl.BlockSpec((BLK, 64), lambda i: (i, 0)),
            pl.BlockSpec((BLK, 1), lambda i: (i, 0)),
            pl.BlockSpec((1, 64), lambda i: (0, 0)),
            pl.BlockSpec((BLK, 16), lambda i: (i, 0)),
            pl.BlockSpec((64, 16), lambda i: (0, 0)),
            pl.BlockSpec((1, 16), lambda i: (0, 0)),
        ],
        out_specs=pl.BlockSpec((16, 16), lambda i: (0, 0)),
        out_shape=jax.ShapeDtypeStruct((16, 16), jnp.float32),
        scratch_shapes=[
            pltpu.VMEM((16, 64), jnp.float32),
            pltpu.VMEM((16, 16), jnp.float32),
        ],
    )(accA, accB, sm2, deg, b2, oh, w3, b3)


_SC_MESH = plsc.VectorSubcoreMesh(core_axis_name="c", subcore_axis_name="s")
_ROWS_PER_TILE = ACC_ROWS // SC_TILES
_TAKE_DN = lax.GatherDimensionNumbers(
    offset_dims=(), collapsed_slice_dims=(0,), start_index_map=(0,))


def _take(v, idx):
    return lax.gather(v, idx[:, None], _TAKE_DN, (1,),
                      mode=lax.GatherScatterMode.PROMISE_IN_BOUNDS)


def _make_edge_sc(tab_dtype, chunk, nchunk):
  @functools.partial(
      pl.kernel,
      out_type=jax.ShapeDtypeStruct((SC_CORES * ACC_ROWS, 32), jnp.float32),
      mesh=_SC_MESH,
      scratch_types=[
          pltpu.VMEM((2, 2, chunk), jnp.int32),
          pltpu.VMEM((2, chunk, 16), jnp.float32),
          pltpu.VMEM((2, chunk, 16), jnp.float32),
          pltpu.VMEM((2, chunk, 256), tab_dtype),
          pltpu.VMEM((2, chunk, 32), jnp.float32),
          pltpu.VMEM((16,), jnp.float32),
          pltpu.VMEM_SHARED((ACC_ROWS, 32), jnp.float32),
          pltpu.SemaphoreType.DMA,
          pltpu.SemaphoreType.DMA,
          pltpu.SemaphoreType.DMA,
          pltpu.SemaphoreType.DMA,
      ],
      compiler_params=pltpu.CompilerParams(use_tc_tiling_on_sc=False,
                                           needs_layout_passes=False),
  )
  def _edge_sc(e2_hbm, hu_hbm, tab_hbm, cc_hbm, out_hbm,
               sd, hus, hud, rows, msg, ccv, acc, semi0, semi1, semg0, semg1):
    cid = lax.axis_index("c")
    sid = lax.axis_index("s")
    wid = sid * SC_CORES + cid
    iota = lax.iota(jnp.int32, 16)
    semi = (semi0, semi1)
    semg = (semg0, semg1)

    zv = jnp.zeros((16,), jnp.float32)

    def _zrow(i, _):
        msg[0, i, pl.ds(0, 16)] = zv
        msg[0, i, pl.ds(16, 16)] = zv
        return 0

    lax.fori_loop(0, chunk, _zrow, 0)

    def _zacc(z, _):
        pltpu.sync_copy(msg.at[0],
                        acc.at[pl.ds(sid * _ROWS_PER_TILE + z * chunk,
                                     chunk)])
        return 0

    lax.fori_loop(0, _ROWS_PER_TILE // chunk, _zacc, 0)

    pltpu.sync_copy(cc_hbm, ccv)
    cc = ccv[...]
    hsplat = [jnp.full((16,), hh, jnp.int32) for hh in range(HEADS)]
    plsc.subcore_barrier()

    tile_base = wid * (nchunk * chunk)

    def _issue_idx(c, b):
        pltpu.async_copy(
            e2_hbm.at[:, pl.ds(tile_base + c * chunk, chunk)], sd.at[b],
            semi[b])

    def _wait_idx(b):
        pltpu.make_async_copy(
            e2_hbm.at[:, pl.ds(tile_base, chunk)], sd.at[b], semi[b]).wait()

    def _issue_gathers(b):
        pltpu.async_copy(hu_hbm.at[sd.at[b, 0]], hus.at[b], semg[b])
        pltpu.async_copy(hu_hbm.at[sd.at[b, 1]], hud.at[b], semg[b])
        pltpu.async_copy(tab_hbm.at[sd.at[b, 0]], rows.at[b], semg[b])

    def _wait_gathers(b):
        pltpu.make_async_copy(hu_hbm.at[sd.at[b, 0]], hus.at[b],
                              semg[b]).wait()
        pltpu.make_async_copy(hu_hbm.at[sd.at[b, 1]], hud.at[b],
                              semg[b]).wait()
        pltpu.make_async_copy(tab_hbm.at[sd.at[b, 0]], rows.at[b],
                              semg[b]).wait()

    himask = jnp.full((16,), -65536, jnp.int32)

    def _compute(b):
        def _quad(i, _):
            for e in (4 * i, 4 * i + 1, 4 * i + 2, 4 * i + 3):
                t = jnp.exp(hus[b, e] - hud[b, e] + cc)
                th = [_take(t, hsplat[hh]) for hh in range(HEADS)]
                s = ((th[0] + th[1]) + (th[2] + th[3])) + (
                    (th[4] + th[5]) + (th[6] + th[7]))
                p0 = []
                p1 = []
                for hh in range(HEADS):
                    if tab_dtype == jnp.bfloat16:
                        vi = plsc.bitcast(rows[b, e, pl.ds(32 * hh, 32)],
                                          jnp.int32)
                        ev = plsc.bitcast(vi << 16, jnp.float32)
                        od = plsc.bitcast(vi & himask, jnp.float32)
                    else:
                        ev = rows[b, e, pl.ds(32 * hh, 16)]
                        od = rows[b, e, pl.ds(32 * hh + 16, 16)]
                    p0.append(th[hh] * ev)
                    p1.append(th[hh] * od)
                m0 = ((p0[0] + p0[1]) + (p0[2] + p0[3])) + (
                    (p0[4] + p0[5]) + (p0[6] + p0[7]))
                m1 = ((p1[0] + p1[1]) + (p1[2] + p1[3])) + (
                    (p1[4] + p1[5]) + (p1[6] + p1[7]))
                msg[b, e, pl.ds(0, 16)] = m0 / s
                msg[b, e, pl.ds(16, 16)] = m1 / s
            return 0

        lax.fori_loop(0, chunk // 4, _quad, 0)

    def _scatter(b):
        pltpu.sync_copy(msg.at[b], acc.at[sd.at[b, 1]], add=True)

    _issue_idx(0, 0)
    _issue_idx(1, 1)
    _wait_idx(0)
    _issue_gathers(0)

    def _body(c, b):
        bn = 1 - b

        @pl.when(c + 1 < nchunk)
        def _():
            _wait_idx(bn)
            _issue_gathers(bn)

        _wait_gathers(b)
        _compute(b)
        _scatter(b)

        @pl.when(c + 2 < nchunk)
        def _():
            _issue_idx(c + 2, b)

    def _pair_body(g, _):
        _body(2 * g, 0)
        _body(2 * g + 1, 1)
        return 0

    lax.fori_loop(0, nchunk // 2, _pair_body, 0)
    plsc.subcore_barrier()

    def _copyout(z, _):
        r0 = sid * _ROWS_PER_TILE + z * ZCH
        pltpu.sync_copy(acc.at[pl.ds(r0, ZCH)],
                        out_hbm.at[pl.ds(cid * ACC_ROWS + r0, ZCH)])
        return 0

    lax.fori_loop(0, ZN, _copyout, 0)

  return _edge_sc


_edge_sc_f32 = _make_edge_sc(jnp.float32, 32, 784)
_edge_sc_bf16 = _make_edge_sc(jnp.bfloat16, 64, 392)


@functools.partial(
    pl.kernel,
    out_type=jax.ShapeDtypeStruct((SC_CORES * ACC_ROWS, 16), jnp.float32),
    mesh=_SC_MESH,
    scratch_types=[
        pltpu.VMEM((2, DCH), jnp.int32),
        pltpu.VMEM((DCH, 16), jnp.float32),
        pltpu.VMEM((ZCH, 16), jnp.float32),
        pltpu.VMEM_SHARED((ACC_ROWS, 16), jnp.float32),
        pltpu.SemaphoreType.DMA,
        pltpu.SemaphoreType.DMA,
    ],
    compiler_params=pltpu.CompilerParams(use_tc_tiling_on_sc=False),
)
def _deg_sc(dst_hbm, out_hbm, didx, ones, zrows, acc, semd0, semd1):
    cid = lax.axis_index("c")
    sid = lax.axis_index("s")
    wid = sid * SC_CORES + cid
    iota = lax.iota(jnp.int32, 16)
    onev = jnp.where(iota == 0, 1.0, 0.0).astype(jnp.float32)
    zv = jnp.zeros((16,), jnp.float32)
    semd = (semd0, semd1)

    def _fill(i, _):
        ones[i, pl.ds(0, 16)] = onev
        return 0

    lax.fori_loop(0, DCH, _fill, 0)

    def _fillz(i, _):
        zrows[i, pl.ds(0, 16)] = zv
        return 0

    lax.fori_loop(0, ZCH, _fillz, 0)

    def _zacc(z, _):
        pltpu.sync_copy(zrows, acc.at[pl.ds(sid * _ROWS_PER_TILE + z * ZCH,
                                            ZCH)])
        return 0

    lax.fori_loop(0, ZN, _zacc, 0)
    plsc.subcore_barrier()

    tile_base = wid * (DNCH * DCH)

    def _dissue(c, b):
        pltpu.async_copy(dst_hbm.at[pl.ds(tile_base + c * DCH, DCH)],
                         didx.at[b], semd[b])

    def _dwait(b):
        pltpu.make_async_copy(dst_hbm.at[pl.ds(tile_base, DCH)], didx.at[b],
                              semd[b]).wait()

    _dissue(0, 0)
    _dissue(1, 1)

    def _dbody(c, b):
        _dwait(b)
        pltpu.sync_copy(ones, acc.at[didx.at[b]], add=True)

        @pl.when(c + 2 < DNCH)
        def _():
            _dissue(c + 2, b)

    def _dpair(g, _):
        _dbody(2 * g, 0)
        _dbody(2 * g + 1, 1)
        return 0

    lax.fori_loop(0, DNCH // 2, _dpair, 0)
    plsc.subcore_barrier()

    def _copyout(z, _):
        r0 = sid * _ROWS_PER_TILE + z * ZCH
        pltpu.sync_copy(acc.at[pl.ds(r0, ZCH)],
                        out_hbm.at[pl.ds(cid * ACC_ROWS + r0, ZCH)])
        return 0

    lax.fori_loop(0, ZN, _copyout, 0)


def kernel(x, edge_index, batch, fc0_w, fc0_b, conv1_weight, conv1_u, conv1_c,
           conv1_bias, conv2_weight, conv2_u, conv2_c, conv2_bias, fc1_w,
           fc1_b):
    f32 = jnp.float32
    q01 = jax.nn.softmax(conv1_c)[None, :]
    q02 = jax.nn.softmax(conv2_c)[None, :]
    cc1 = jnp.concatenate([conv1_c, jnp.full((8,), -1e30, f32)])
    cc2 = jnp.concatenate([conv2_c, jnp.full((8,), -1e30, f32)])

    perm32 = np.concatenate([np.arange(0, 32, 2), np.arange(1, 32, 2)])
    perm64 = np.concatenate([perm32, 32 + perm32])
    p64 = jnp.asarray(np.eye(64, dtype=np.float32)[perm64].T)

    src = jnp.concatenate(
        [edge_index[0], jnp.zeros((EPAD - E,), jnp.int32)])
    dst = jnp.concatenate(
        [edge_index[1], jnp.full((EPAD - E,), DUMMY_DST, jnp.int32)])
    e2 = jnp.stack([src, dst])

    hw1, hu1, sm1 = _dense1(x, fc0_w, fc0_b[None, :], conv1_weight, conv1_u,
                            q01)
    hu1p = jnp.zeros((HU_ROWS, 16), f32).at[0:N].set(hu1)
    dega = _deg_sc(dst).reshape(2, ACC_ROWS, 16)
    acc1 = _edge_sc_f32(e2, hu1p, hw1, cc1).reshape(2, ACC_ROWS, 32)

    ta, tb, hu2, sm2, deg = _dense2(acc1, dega, sm1, conv1_bias[None, :],
                                    conv2_weight, conv2_u, q02, p64)
    hu2p = jnp.zeros((HU_ROWS, 16), f32).at[0:N].set(hu2)
    accA = _edge_sc_bf16(e2, hu2p, ta, cc2).reshape(2, ACC_ROWS, 32)
    accB = _edge_sc_bf16(e2, hu2p, tb, cc2).reshape(2, ACC_ROWS, 32)

    oh = (batch[:, None] == jnp.arange(16, dtype=jnp.int32)[None, :]
          ).astype(f32)
    out16 = _dense3(accA, accB, sm2, deg, conv2_bias[perm64][None, :], oh,
                    jnp.zeros((64, 16), f32).at[:, 0:10].set(fc1_w[perm64]),
                    jnp.zeros((1, 16), f32).at[0, 0:10].set(fc1_b))
    return out16[:, 0:10]


# speedup vs baseline: 5.8977x; 1.2145x over previous
import functools

import jax
import jax.numpy as jnp
import numpy as np
from jax import lax
from jax.experimental import pallas as pl
from jax.experimental.pallas import tpu as pltpu
from jax.experimental.pallas import tpu_sc as plsc

N = 50000
E = 800000
HEADS = 8
BLK = 2000
NBLK = N // BLK

SC_CORES = 2
SC_TILES = 16
SC_WORKERS = SC_CORES * SC_TILES
CHUNK = 64
NCHUNK = 392
EPAD = SC_WORKERS * CHUNK * NCHUNK
DCH = 128
DNCH = EPAD // (SC_WORKERS * DCH)
ACC_ROWS = 50176
ZCH = 64
ZN = ACC_ROWS // SC_TILES // ZCH
HU_ROWS = 50016
DUMMY_DST = N + 8


def _dense1_body(x_ref, w0_ref, b0_ref, w1_ref, u1_ref, q0_ref,
                 hw_ref, hu_ref, sm_ref):
    h = jnp.maximum(jnp.dot(x_ref[...], w0_ref[...],
                            preferred_element_type=jnp.float32)
                    + b0_ref[...], 0.0)
    hw = jnp.dot(h, w1_ref[...], preferred_element_type=jnp.float32)
    hw_ref[...] = hw
    hu = jnp.dot(h, u1_ref[...], preferred_element_type=jnp.float32)
    hu_ref[...] = jnp.concatenate([hu, jnp.zeros_like(hu)], axis=1)
    q0 = q0_ref[...]
    sm = jnp.zeros((x_ref.shape[0], 32), jnp.float32)
    for hh in range(HEADS):
        sm = sm + q0[0:1, hh:hh + 1] * hw[:, 32 * hh:32 * hh + 32]
    sm_ref[...] = sm


def _dense1(x, w0, b0, w1, u1, q0):
    return pl.pallas_call(
        _dense1_body,
        grid=(NBLK,),
        in_specs=[
            pl.BlockSpec((BLK, 128), lambda i: (i, 0)),
            pl.BlockSpec((128, 16), lambda i: (0, 0)),
            pl.BlockSpec((1, 16), lambda i: (0, 0)),
            pl.BlockSpec((16, 256), lambda i: (0, 0)),
            pl.BlockSpec((16, 8), lambda i: (0, 0)),
            pl.BlockSpec((1, 8), lambda i: (0, 0)),
        ],
        out_specs=[
            pl.BlockSpec((BLK, 256), lambda i: (i, 0)),
            pl.BlockSpec((BLK, 16), lambda i: (i, 0)),
            pl.BlockSpec((BLK, 32), lambda i: (i, 0)),
        ],
        out_shape=[
            jax.ShapeDtypeStruct((N, 256), jnp.float32),
            jax.ShapeDtypeStruct((N, 16), jnp.float32),
            jax.ShapeDtypeStruct((N, 32), jnp.float32),
        ],
    )(x, w0, b0, w1, u1, q0)


def _dense2_body(acc_ref, dg_ref, sm1_ref, b1_ref, w2_ref, u2_ref, q0_ref,
                 p64_ref, ta_ref, tb_ref, hu_ref, sm2_ref, deg_ref):
    a = acc_ref[0] + acc_ref[1] + sm1_ref[...]
    deg = dg_ref[0, :, 0:1] + dg_ref[1, :, 0:1] + 1.0
    deg_ref[...] = deg
    out1 = jnp.maximum(a / deg + b1_ref[...], 0.0)
    hw2 = jnp.dot(out1, w2_ref[...], preferred_element_type=jnp.float32)
    q0 = q0_ref[...]
    sm = jnp.zeros((out1.shape[0], 64), jnp.float32)
    for hh in range(HEADS):
        blk = hw2[:, 64 * hh:64 * hh + 64]
        sm = sm + q0[0:1, hh:hh + 1] * blk
        ta_ref[:, 32 * hh:32 * hh + 32] = blk[:, 0:32].astype(jnp.bfloat16)
        tb_ref[:, 32 * hh:32 * hh + 32] = blk[:, 32:64].astype(jnp.bfloat16)
    sm2_ref[...] = jnp.dot(sm, p64_ref[...],
                           preferred_element_type=jnp.float32)
    hu = jnp.dot(out1, u2_ref[...], preferred_element_type=jnp.float32)
    hu_ref[...] = jnp.concatenate([hu, jnp.zeros_like(hu)], axis=1)


def _dense2(acc1, dega, sm1, b1, w2, u2, q0, p64):
    return pl.pallas_call(
        _dense2_body,
        grid=(NBLK,),
        in_specs=[
            pl.BlockSpec((2, BLK, 32), lambda i: (0, i, 0)),
            pl.BlockSpec((2, BLK, 16), lambda i: (0, i, 0)),
            pl.BlockSpec((BLK, 32), lambda i: (i, 0)),
            pl.BlockSpec((1, 32), lambda i: (0, 0)),
            pl.BlockSpec((32, 512), lambda i: (0, 0)),
            pl.BlockSpec((32, 8), lambda i: (0, 0)),
            pl.BlockSpec((1, 8), lambda i: (0, 0)),
            pl.BlockSpec((64, 64), lambda i: (0, 0)),
        ],
        out_specs=[
            pl.BlockSpec((BLK, 256), lambda i: (i, 0)),
            pl.BlockSpec((BLK, 256), lambda i: (i, 0)),
            pl.BlockSpec((BLK, 16), lambda i: (i, 0)),
            pl.BlockSpec((BLK, 64), lambda i: (i, 0)),
            pl.BlockSpec((BLK, 1), lambda i: (i, 0)),
        ],
        out_shape=[
            jax.ShapeDtypeStruct((N, 256), jnp.bfloat16),
            jax.ShapeDtypeStruct((N, 256), jnp.bfloat16),
            jax.ShapeDtypeStruct((N, 16), jnp.float32),
            jax.ShapeDtypeStruct((N, 64), jnp.float32),
            jax.ShapeDtypeStruct((N, 1), jnp.float32),
        ],
    )(acc1, dega, sm1, b1, w2, u2, q0, p64)


def _dense3_body(aa_ref, ab_ref, sm2_ref, deg_ref, b2_ref, oh_ref,
                 w3_ref, b3_ref, out_ref, sums_ref, cnt_ref):
    i = pl.program_id(0)

    @pl.when(i == 0)
    def _():
        sums_ref[...] = jnp.zeros_like(sums_ref)
        cnt_ref[...] = jnp.zeros_like(cnt_ref)

    deg = deg_ref[...]
    a = (aa_ref[0] + aa_ref[1] + sm2_ref[:, 0:32]) / deg
    b = (ab_ref[0] + ab_ref[1] + sm2_ref[:, 32:64]) / deg
    out2 = jnp.maximum(jnp.concatenate([a, b], axis=1) + b2_ref[...], 0.0)
    oh = oh_ref[...]
    sums_ref[...] += lax.dot_general(oh, out2, (((0,), (0,)), ((), ())),
                                     preferred_element_type=jnp.float32)
    cnt_ref[...] += lax.dot_general(oh, jnp.ones_like(oh),
                                    (((0,), (0,)), ((), ())),
                                    preferred_element_type=jnp.float32)

    @pl.when(i == NBLK - 1)
    def _():
        g = sums_ref[...] / jnp.maximum(cnt_ref[:, 0:1], 1.0)
        out_ref[...] = jnp.dot(g, w3_ref[...],
                               preferred_element_type=jnp.float32) + b3_ref[...]


def _dense3(accA, accB, sm2, deg, b2, oh, w3, b3):
    return pl.pallas_call(
        _dense3_body,
        grid=(NBLK,),
        in_specs=[
            pl.BlockSpec((2, BLK, 32), lambda i: (0, i, 0)),
            pl.BlockSpec((2, BLK, 32), lambda i: (0, i, 0)),
            pl.BlockSpec((BLK, 64), lambda i: (i, 0)),
            pl.BlockSpec((BLK, 1), lambda i: (i, 0)),
            pl.BlockSpec((1, 64), lambda i: (0, 0)),
            pl.BlockSpec((BLK, 16), lambda i: (i, 0)),
            pl.BlockSpec((64, 16), lambda i: (0, 0)),
            pl.BlockSpec((1, 16), lambda i: (0, 0)),
        ],
        out_specs=pl.BlockSpec((16, 16), lambda i: (0, 0)),
        out_shape=jax.ShapeDtypeStruct((16, 16), jnp.float32),
        scratch_shapes=[
            pltpu.VMEM((16, 64), jnp.float32),
            pltpu.VMEM((16, 16), jnp.float32),
        ],
    )(accA, accB, sm2, deg, b2, oh, w3, b3)


_SC_MESH = plsc.VectorSubcoreMesh(core_axis_name="c", subcore_axis_name="s")
_ROWS_PER_TILE = ACC_ROWS // SC_TILES
_TAKE_DN = lax.GatherDimensionNumbers(
    offset_dims=(), collapsed_slice_dims=(0,), start_index_map=(0,))


def _take(v, idx):
    return lax.gather(v, idx[:, None], _TAKE_DN, (1,),
                      mode=lax.GatherScatterMode.PROMISE_IN_BOUNDS)


def _make_edge_sc(tab_dtype, chunk, nchunk):
  @functools.partial(
      pl.kernel,
      out_type=jax.ShapeDtypeStruct((SC_CORES * ACC_ROWS, 32), jnp.float32),
      mesh=_SC_MESH,
      scratch_types=[
          pltpu.VMEM((4, 2, chunk), jnp.int32),
          pltpu.VMEM((2, chunk, 16), jnp.float32),
          pltpu.VMEM((2, chunk, 16), jnp.float32),
          pltpu.VMEM((2, chunk, 256), tab_dtype),
          pltpu.VMEM((2, chunk, 32), jnp.float32),
          pltpu.VMEM((16,), jnp.float32),
          pltpu.VMEM_SHARED((ACC_ROWS, 32), jnp.float32),
          [pltpu.SemaphoreType.DMA] * 4,
          [pltpu.SemaphoreType.DMA] * 2,
          [pltpu.SemaphoreType.DMA] * 2,
      ],
      compiler_params=pltpu.CompilerParams(use_tc_tiling_on_sc=False,
                                           needs_layout_passes=False),
  )
  def _edge_sc(e2_hbm, hu_hbm, tab_hbm, cc_hbm, out_hbm,
               sd, hus, hud, rows, msg, ccv, acc, semi, semg, sems):
    cid = lax.axis_index("c")
    sid = lax.axis_index("s")
    wid = sid * SC_CORES + cid

    zv = jnp.zeros((16,), jnp.float32)

    def _zrow(i, _):
        msg[0, i, pl.ds(0, 16)] = zv
        msg[0, i, pl.ds(16, 16)] = zv
        return 0

    lax.fori_loop(0, chunk, _zrow, 0)

    def _zacc(z, _):
        pltpu.sync_copy(msg.at[0],
                        acc.at[pl.ds(sid * _ROWS_PER_TILE + z * chunk,
                                     chunk)])
        return 0

    lax.fori_loop(0, _ROWS_PER_TILE // chunk, _zacc, 0)

    pltpu.sync_copy(cc_hbm, ccv)
    cc = ccv[...]
    hsplat = [jnp.full((16,), hh, jnp.int32) for hh in range(HEADS)]
    plsc.subcore_barrier()

    tile_base = wid * (nchunk * chunk)

    def _issue_idx(c, q):
        pltpu.async_copy(
            e2_hbm.at[:, pl.ds(tile_base + c * chunk, chunk)], sd.at[q],
            semi[q])

    def _wait_idx(q):
        pltpu.make_async_copy(
            e2_hbm.at[:, pl.ds(tile_base, chunk)], sd.at[q], semi[q]).wait()

    def _issue_gathers(b, q):
        pltpu.async_copy(hu_hbm.at[sd.at[q, 0]], hus.at[b], semg[b])
        pltpu.async_copy(hu_hbm.at[sd.at[q, 1]], hud.at[b], semg[b])
        pltpu.async_copy(tab_hbm.at[sd.at[q, 0]], rows.at[b], semg[b])

    def _wait_gathers(b, q):
        pltpu.make_async_copy(hu_hbm.at[sd.at[q, 0]], hus.at[b],
                              semg[b]).wait()
        pltpu.make_async_copy(hu_hbm.at[sd.at[q, 1]], hud.at[b],
                              semg[b]).wait()
        pltpu.make_async_copy(tab_hbm.at[sd.at[q, 0]], rows.at[b],
                              semg[b]).wait()

    def _issue_scatter(b, q):
        pltpu.async_copy(msg.at[b], acc.at[sd.at[q, 1]], sems[b], add=True)

    def _wait_scatter(b, q):
        pltpu.make_async_copy(msg.at[b], acc.at[sd.at[q, 1]], sems[b]).wait()

    himask = jnp.full((16,), -65536, jnp.int32)

    def _compute(b):
        def _quad(i, _):
            for e in (4 * i, 4 * i + 1, 4 * i + 2, 4 * i + 3):
                t = jnp.exp(hus[b, e] - hud[b, e] + cc)
                th = [_take(t, hsplat[hh]) for hh in range(HEADS)]
                s = ((th[0] + th[1]) + (th[2] + th[3])) + (
                    (th[4] + th[5]) + (th[6] + th[7]))
                p0 = []
                p1 = []
                for hh in range(HEADS):
                    if tab_dtype == jnp.bfloat16:
                        vi = plsc.bitcast(rows[b, e, pl.ds(32 * hh, 32)],
                                          jnp.int32)
                        ev = plsc.bitcast(vi << 16, jnp.float32)
                        od = plsc.bitcast(vi & himask, jnp.float32)
                    else:
                        ev = rows[b, e, pl.ds(32 * hh, 16)]
                        od = rows[b, e, pl.ds(32 * hh + 16, 16)]
                    p0.append(th[hh] * ev)
                    p1.append(th[hh] * od)
                m0 = ((p0[0] + p0[1]) + (p0[2] + p0[3])) + (
                    (p0[4] + p0[5]) + (p0[6] + p0[7]))
                m1 = ((p1[0] + p1[1]) + (p1[2] + p1[3])) + (
                    (p1[4] + p1[5]) + (p1[6] + p1[7]))
                msg[b, e, pl.ds(0, 16)] = m0 / s
                msg[b, e, pl.ds(16, 16)] = m1 / s
            return 0

        lax.fori_loop(0, chunk // 4, _quad, 0)

    _issue_idx(0, 0)
    _issue_idx(1, 1)
    _wait_idx(0)
    _issue_gathers(0, 0)

    def _body(g, k):
        c = 4 * g + k
        b = k % 2
        bn = 1 - b

        @pl.when(c + 1 < nchunk)
        def _():
            _wait_idx((k + 1) % 4)
            _issue_gathers(bn, (k + 1) % 4)

        _wait_gathers(b, k)

        @pl.when(c >= 2)
        def _():
            _wait_scatter(b, (k + 2) % 4)

        @pl.when(c + 2 < nchunk)
        def _():
            _issue_idx(c + 2, (k + 2) % 4)

        _compute(b)
        _issue_scatter(b, k)

    def _quad_body(g, _):
        for k in range(4):
            _body(g, k)
        return 0

    lax.fori_loop(0, nchunk // 4, _quad_body, 0)
    _wait_scatter(0, (nchunk - 2) % 4)
    _wait_scatter(1, (nchunk - 1) % 4)
    plsc.subcore_barrier()

    def _copyout(z, _):
        r0 = sid * _ROWS_PER_TILE + z * ZCH
        pltpu.sync_copy(acc.at[pl.ds(r0, ZCH)],
                        out_hbm.at[pl.ds(cid * ACC_ROWS + r0, ZCH)])
        return 0

    lax.fori_loop(0, ZN, _copyout, 0)

  return _edge_sc


_edge_sc_f32 = _make_edge_sc(jnp.float32, 32, 784)
_edge_sc_bf16 = _make_edge_sc(jnp.bfloat16, 64, 392)


@functools.partial(
    pl.kernel,
    out_type=jax.ShapeDtypeStruct((SC_CORES * ACC_ROWS, 16), jnp.float32),
    mesh=_SC_MESH,
    scratch_types=[
        pltpu.VMEM((2, DCH), jnp.int32),
        pltpu.VMEM((DCH, 16), jnp.float32),
        pltpu.VMEM((ZCH, 16), jnp.float32),
        pltpu.VMEM_SHARED((ACC_ROWS, 16), jnp.float32),
        pltpu.SemaphoreType.DMA,
        pltpu.SemaphoreType.DMA,
    ],
    compiler_params=pltpu.CompilerParams(use_tc_tiling_on_sc=False),
)
def _deg_sc(dst_hbm, out_hbm, didx, ones, zrows, acc, semd0, semd1):
    cid = lax.axis_index("c")
    sid = lax.axis_index("s")
    wid = sid * SC_CORES + cid
    iota = lax.iota(jnp.int32, 16)
    onev = jnp.where(iota == 0, 1.0, 0.0).astype(jnp.float32)
    zv = jnp.zeros((16,), jnp.float32)
    semd = (semd0, semd1)

    def _fill(i, _):
        ones[i, pl.ds(0, 16)] = onev
        return 0

    lax.fori_loop(0, DCH, _fill, 0)

    def _fillz(i, _):
        zrows[i, pl.ds(0, 16)] = zv
        return 0

    lax.fori_loop(0, ZCH, _fillz, 0)

    def _zacc(z, _):
        pltpu.sync_copy(zrows, acc.at[pl.ds(sid * _ROWS_PER_TILE + z * ZCH,
                                            ZCH)])
        return 0

    lax.fori_loop(0, ZN, _zacc, 0)
    plsc.subcore_barrier()

    tile_base = wid * (DNCH * DCH)

    def _dissue(c, b):
        pltpu.async_copy(dst_hbm.at[pl.ds(tile_base + c * DCH, DCH)],
                         didx.at[b], semd[b])

    def _dwait(b):
        pltpu.make_async_copy(dst_hbm.at[pl.ds(tile_base, DCH)], didx.at[b],
                              semd[b]).wait()

    _dissue(0, 0)
    _dissue(1, 1)

    def _dbody(c, b):
        _dwait(b)
        pltpu.sync_copy(ones, acc.at[didx.at[b]], add=True)

        @pl.when(c + 2 < DNCH)
        def _():
            _dissue(c + 2, b)

    def _dpair(g, _):
        _dbody(2 * g, 0)
        _dbody(2 * g + 1, 1)
        return 0

    lax.fori_loop(0, DNCH // 2, _dpair, 0)
    plsc.subcore_barrier()

    def _copyout(z, _):
        r0 = sid * _ROWS_PER_TILE + z * ZCH
        pltpu.sync_copy(acc.at[pl.ds(r0, ZCH)],
                        out_hbm.at[pl.ds(cid * ACC_ROWS + r0, ZCH)])
        return 0

    lax.fori_loop(0, ZN, _copyout, 0)


def kernel(x, edge_index, batch, fc0_w, fc0_b, conv1_weight, conv1_u, conv1_c,
           conv1_bias, conv2_weight, conv2_u, conv2_c, conv2_bias, fc1_w,
           fc1_b):
    f32 = jnp.float32
    q01 = jax.nn.softmax(conv1_c)[None, :]
    q02 = jax.nn.softmax(conv2_c)[None, :]
    cc1 = jnp.concatenate([conv1_c, jnp.full((8,), -1e30, f32)])
    cc2 = jnp.concatenate([conv2_c, jnp.full((8,), -1e30, f32)])

    perm32 = np.concatenate([np.arange(0, 32, 2), np.arange(1, 32, 2)])
    perm64 = np.concatenate([perm32, 32 + perm32])
    p64 = jnp.asarray(np.eye(64, dtype=np.float32)[perm64].T)

    src = jnp.concatenate(
        [edge_index[0], jnp.zeros((EPAD - E,), jnp.int32)])
    dst = jnp.concatenate(
        [edge_index[1], jnp.full((EPAD - E,), DUMMY_DST, jnp.int32)])
    e2 = jnp.stack([src, dst])

    hw1, hu1, sm1 = _dense1(x, fc0_w, fc0_b[None, :], conv1_weight, conv1_u,
                            q01)
    hu1p = jnp.zeros((HU_ROWS, 16), f32).at[0:N].set(hu1)
    dega = _deg_sc(dst).reshape(2, ACC_ROWS, 16)
    acc1 = _edge_sc_f32(e2, hu1p, hw1, cc1).reshape(2, ACC_ROWS, 32)

    ta, tb, hu2, sm2, deg = _dense2(acc1, dega, sm1, conv1_bias[None, :],
                                    conv2_weight, conv2_u, q02, p64)
    hu2p = jnp.zeros((HU_ROWS, 16), f32).at[0:N].set(hu2)
    accA = _edge_sc_bf16(e2, hu2p, ta, cc2).reshape(2, ACC_ROWS, 32)
    accB = _edge_sc_bf16(e2, hu2p, tb, cc2).reshape(2, ACC_ROWS, 32)

    oh = (batch[:, None] == jnp.arange(16, dtype=jnp.int32)[None, :]
          ).astype(f32)
    out16 = _dense3(accA, accB, sm2, deg, conv2_bias[perm64][None, :], oh,
                    jnp.zeros((64, 16), f32).at[:, 0:10].set(fc1_w[perm64]),
                    jnp.zeros((1, 16), f32).at[0, 0:10].set(fc1_b))
    return out16[:, 0:10]


# speedup vs baseline: 7.1156x; 1.2065x over previous
import functools

import jax
import jax.numpy as jnp
import numpy as np
from jax import lax
from jax.experimental import pallas as pl
from jax.experimental.pallas import tpu as pltpu
from jax.experimental.pallas import tpu_sc as plsc

N = 50000
E = 800000
HEADS = 8
BLK = 2000
NBLK = N // BLK

SC_CORES = 2
SC_TILES = 16
SC_WORKERS = SC_CORES * SC_TILES
CHUNK = 64
NCHUNK = 392
EPAD = SC_WORKERS * CHUNK * NCHUNK
DCH = 128
DNCH = EPAD // (SC_WORKERS * DCH)
ACC_ROWS = 50176
ZCH = 64
ZN = ACC_ROWS // SC_TILES // ZCH
HU_ROWS = 50016
DUMMY_DST = N + 8


def _dense1_body(x_ref, w0_ref, b0_ref, w1_ref, u1_ref, q0_ref,
                 hw_ref, hu_ref, sm_ref):
    h = jnp.maximum(jnp.dot(x_ref[...], w0_ref[...],
                            preferred_element_type=jnp.float32)
                    + b0_ref[...], 0.0)
    hw = jnp.dot(h, w1_ref[...], preferred_element_type=jnp.float32)
    hw_ref[...] = hw
    hu = jnp.dot(h, u1_ref[...], preferred_element_type=jnp.float32)
    hu_ref[...] = jnp.concatenate([hu, jnp.zeros_like(hu)], axis=1)
    q0 = q0_ref[...]
    sm = jnp.zeros((x_ref.shape[0], 32), jnp.float32)
    for hh in range(HEADS):
        sm = sm + q0[0:1, hh:hh + 1] * hw[:, 32 * hh:32 * hh + 32]
    sm_ref[...] = sm


def _dense1(x, w0, b0, w1, u1, q0):
    return pl.pallas_call(
        _dense1_body,
        grid=(NBLK,),
        in_specs=[
            pl.BlockSpec((BLK, 128), lambda i: (i, 0)),
            pl.BlockSpec((128, 16), lambda i: (0, 0)),
            pl.BlockSpec((1, 16), lambda i: (0, 0)),
            pl.BlockSpec((16, 256), lambda i: (0, 0)),
            pl.BlockSpec((16, 8), lambda i: (0, 0)),
            pl.BlockSpec((1, 8), lambda i: (0, 0)),
        ],
        out_specs=[
            pl.BlockSpec((BLK, 256), lambda i: (i, 0)),
            pl.BlockSpec((BLK, 16), lambda i: (i, 0)),
            pl.BlockSpec((BLK, 32), lambda i: (i, 0)),
        ],
        out_shape=[
            jax.ShapeDtypeStruct((N, 256), jnp.float32),
            jax.ShapeDtypeStruct((N, 16), jnp.float32),
            jax.ShapeDtypeStruct((N, 32), jnp.float32),
        ],
    )(x, w0, b0, w1, u1, q0)


def _dense2_body(acc_ref, dg_ref, sm1_ref, b1_ref, w2_ref, u2_ref, q0_ref,
                 tw_ref, hu_ref, sm2_ref, deg_ref):
    a = acc_ref[0] + acc_ref[1] + sm1_ref[...]
    deg = dg_ref[0, :, 0:1] + dg_ref[1, :, 0:1] + 1.0
    deg_ref[...] = deg
    out1 = jnp.maximum(a / deg + b1_ref[...], 0.0)
    hw2 = jnp.dot(out1, w2_ref[...], preferred_element_type=jnp.float32)
    tw_ref[...] = hw2.astype(jnp.bfloat16)
    q0 = q0_ref[...]
    sm = jnp.zeros((out1.shape[0], 64), jnp.float32)
    for hh in range(HEADS):
        sm = sm + q0[0:1, hh:hh + 1] * hw2[:, 64 * hh:64 * hh + 64]
    sm2_ref[...] = sm
    hu = jnp.dot(out1, u2_ref[...], preferred_element_type=jnp.float32)
    hu_ref[...] = jnp.concatenate([hu, jnp.zeros_like(hu)], axis=1)


def _dense2(acc1, dega, sm1, b1, w2, u2, q0):
    return pl.pallas_call(
        _dense2_body,
        grid=(NBLK,),
        in_specs=[
            pl.BlockSpec((2, BLK, 32), lambda i: (0, i, 0)),
            pl.BlockSpec((2, BLK, 16), lambda i: (0, i, 0)),
            pl.BlockSpec((BLK, 32), lambda i: (i, 0)),
            pl.BlockSpec((1, 32), lambda i: (0, 0)),
            pl.BlockSpec((32, 512), lambda i: (0, 0)),
            pl.BlockSpec((32, 8), lambda i: (0, 0)),
            pl.BlockSpec((1, 8), lambda i: (0, 0)),
        ],
        out_specs=[
            pl.BlockSpec((BLK, 512), lambda i: (i, 0)),
            pl.BlockSpec((BLK, 16), lambda i: (i, 0)),
            pl.BlockSpec((BLK, 64), lambda i: (i, 0)),
            pl.BlockSpec((BLK, 1), lambda i: (i, 0)),
        ],
        out_shape=[
            jax.ShapeDtypeStruct((N, 512), jnp.bfloat16),
            jax.ShapeDtypeStruct((N, 16), jnp.float32),
            jax.ShapeDtypeStruct((N, 64), jnp.float32),
            jax.ShapeDtypeStruct((N, 1), jnp.float32),
        ],
    )(acc1, dega, sm1, b1, w2, u2, q0)


def _dense3_body(aa_ref, sm2_ref, deg_ref, b2_ref, oh_ref,
                 w3_ref, b3_ref, out_ref, sums_ref, cnt_ref):
    i = pl.program_id(0)

    @pl.when(i == 0)
    def _():
        sums_ref[...] = jnp.zeros_like(sums_ref)
        cnt_ref[...] = jnp.zeros_like(cnt_ref)

    deg = deg_ref[...]
    a = (aa_ref[0].astype(jnp.float32) + aa_ref[1].astype(jnp.float32)
         + sm2_ref[...]) / deg
    out2 = jnp.maximum(a + b2_ref[...], 0.0)
    oh = oh_ref[...]
    sums_ref[...] += lax.dot_general(oh, out2, (((0,), (0,)), ((), ())),
                                     preferred_element_type=jnp.float32)
    cnt_ref[...] += lax.dot_general(oh, jnp.ones_like(oh),
                                    (((0,), (0,)), ((), ())),
                                    preferred_element_type=jnp.float32)

    @pl.when(i == NBLK - 1)
    def _():
        g = sums_ref[...] / jnp.maximum(cnt_ref[:, 0:1], 1.0)
        out_ref[...] = jnp.dot(g, w3_ref[...],
                               preferred_element_type=jnp.float32) + b3_ref[...]


def _dense3(accA, sm2, deg, b2, oh, w3, b3):
    return pl.pallas_call(
        _dense3_body,
        grid=(NBLK,),
        in_specs=[
            pl.BlockSpec((2, BLK, 64), lambda i: (0, i, 0)),
            pl.BlockSpec((BLK, 64), lambda i: (i, 0)),
            pl.BlockSpec((BLK, 1), lambda i: (i, 0)),
            pl.BlockSpec((1, 64), lambda i: (0, 0)),
            pl.BlockSpec((BLK, 16), lambda i: (i, 0)),
            pl.BlockSpec((64, 16), lambda i: (0, 0)),
            pl.BlockSpec((1, 16), lambda i: (0, 0)),
        ],
        out_specs=pl.BlockSpec((16, 16), lambda i: (0, 0)),
        out_shape=jax.ShapeDtypeStruct((16, 16), jnp.float32),
        scratch_shapes=[
            pltpu.VMEM((16, 64), jnp.float32),
            pltpu.VMEM((16, 16), jnp.float32),
        ],
    )(accA, sm2, deg, b2, oh, w3, b3)


_SC_MESH = plsc.VectorSubcoreMesh(core_axis_name="c", subcore_axis_name="s")
_ROWS_PER_TILE = ACC_ROWS // SC_TILES
_TAKE_DN = lax.GatherDimensionNumbers(
    offset_dims=(), collapsed_slice_dims=(0,), start_index_map=(0,))


def _take(v, idx):
    return lax.gather(v, idx[:, None], _TAKE_DN, (1,),
                      mode=lax.GatherScatterMode.PROMISE_IN_BOUNDS)


def _make_edge_sc(tab_dtype, chunk, nchunk, wide=False):
  @functools.partial(
      pl.kernel,
      out_type=jax.ShapeDtypeStruct(
          (SC_CORES * ACC_ROWS, 64 if wide else 32),
          jnp.bfloat16 if wide else jnp.float32),
      mesh=_SC_MESH,
      scratch_types=[
          pltpu.VMEM((4, 2, chunk), jnp.int32),
          pltpu.VMEM((2, chunk, 16), jnp.float32),
          pltpu.VMEM((2, chunk, 16), jnp.float32),
          pltpu.VMEM((2, chunk, 512 if wide else 256), tab_dtype),
          pltpu.VMEM((2, chunk, 64 if wide else 32),
                     jnp.bfloat16 if wide else jnp.float32),
          pltpu.VMEM((16,), jnp.float32),
          pltpu.VMEM_SHARED((ACC_ROWS, 64 if wide else 32),
                            jnp.bfloat16 if wide else jnp.float32),
          [pltpu.SemaphoreType.DMA] * 4,
          [pltpu.SemaphoreType.DMA] * 2,
          [pltpu.SemaphoreType.DMA] * 2,
      ],
      compiler_params=pltpu.CompilerParams(use_tc_tiling_on_sc=False,
                                           needs_layout_passes=False),
  )
  def _edge_sc(e2_hbm, hu_hbm, tab_hbm, cc_hbm, out_hbm,
               sd, hus, hud, rows, msg, ccv, acc, semi, semg, sems):
    cid = lax.axis_index("c")
    sid = lax.axis_index("s")
    wid = sid * SC_CORES + cid

    if wide:
        zv = jnp.zeros((32,), jnp.bfloat16)

        def _zrow(i, _):
            msg[0, i, pl.ds(0, 32)] = zv
            msg[0, i, pl.ds(32, 32)] = zv
            return 0
    else:
        zv = jnp.zeros((16,), jnp.float32)

        def _zrow(i, _):
            msg[0, i, pl.ds(0, 16)] = zv
            msg[0, i, pl.ds(16, 16)] = zv
            return 0

    lax.fori_loop(0, chunk, _zrow, 0)

    def _zacc(z, _):
        pltpu.sync_copy(msg.at[0],
                        acc.at[pl.ds(sid * _ROWS_PER_TILE + z * chunk,
                                     chunk)])
        return 0

    lax.fori_loop(0, _ROWS_PER_TILE // chunk, _zacc, 0)

    pltpu.sync_copy(cc_hbm, ccv)
    cc = ccv[...]
    hsplat = [jnp.full((16,), hh, jnp.int32) for hh in range(HEADS)]
    plsc.subcore_barrier()

    tile_base = wid * (nchunk * chunk)

    def _issue_idx(c, q):
        pltpu.async_copy(
            e2_hbm.at[:, pl.ds(tile_base + c * chunk, chunk)], sd.at[q],
            semi[q])

    def _wait_idx(q):
        pltpu.make_async_copy(
            e2_hbm.at[:, pl.ds(tile_base, chunk)], sd.at[q], semi[q]).wait()

    def _issue_gathers(b, q):
        pltpu.async_copy(hu_hbm.at[sd.at[q, 0]], hus.at[b], semg[b])
        pltpu.async_copy(hu_hbm.at[sd.at[q, 1]], hud.at[b], semg[b])
        pltpu.async_copy(tab_hbm.at[sd.at[q, 0]], rows.at[b], semg[b])

    def _wait_gathers(b, q):
        pltpu.make_async_copy(hu_hbm.at[sd.at[q, 0]], hus.at[b],
                              semg[b]).wait()
        pltpu.make_async_copy(hu_hbm.at[sd.at[q, 1]], hud.at[b],
                              semg[b]).wait()
        pltpu.make_async_copy(tab_hbm.at[sd.at[q, 0]], rows.at[b],
                              semg[b]).wait()

    def _issue_scatter(b, q):
        pltpu.async_copy(msg.at[b], acc.at[sd.at[q, 1]], sems[b], add=True)

    def _wait_scatter(b, q):
        pltpu.make_async_copy(msg.at[b], acc.at[sd.at[q, 1]], sems[b]).wait()

    himask = jnp.full((16,), -65536, jnp.int32)

    def _tree8(p):
        return ((p[0] + p[1]) + (p[2] + p[3])) + ((p[4] + p[5]) + (p[6] + p[7]))

    def _compute(b):
        def _quad(i, _):
            for e in (4 * i, 4 * i + 1, 4 * i + 2, 4 * i + 3):
                t = jnp.exp(hus[b, e] - hud[b, e] + cc)
                th = [_take(t, hsplat[hh]) for hh in range(HEADS)]
                s = _tree8(th)
                if wide:
                    pe0, po0, pe1, po1 = [], [], [], []
                    for hh in range(HEADS):
                        vi0 = plsc.bitcast(rows[b, e, pl.ds(64 * hh, 32)],
                                           jnp.int32)
                        vi1 = plsc.bitcast(
                            rows[b, e, pl.ds(64 * hh + 32, 32)], jnp.int32)
                        pe0.append(th[hh]
                                   * plsc.bitcast(vi0 << 16, jnp.float32))
                        po0.append(th[hh]
                                   * plsc.bitcast(vi0 & himask, jnp.float32))
                        pe1.append(th[hh]
                                   * plsc.bitcast(vi1 << 16, jnp.float32))
                        po1.append(th[hh]
                                   * plsc.bitcast(vi1 & himask, jnp.float32))
                    rs = 1.0 / s
                    w0 = plsc.pack(_tree8(pe0) * rs, _tree8(po0) * rs,
                                   format=plsc.PackFormat.INTERLEAVED)
                    w1 = plsc.pack(_tree8(pe1) * rs, _tree8(po1) * rs,
                                   format=plsc.PackFormat.INTERLEAVED)
                    msg[b, e, pl.ds(0, 32)] = w0
                    msg[b, e, pl.ds(32, 32)] = w1
                else:
                    p0 = []
                    p1 = []
                    for hh in range(HEADS):
                        if tab_dtype == jnp.bfloat16:
                            vi = plsc.bitcast(rows[b, e, pl.ds(32 * hh, 32)],
                                              jnp.int32)
                            ev = plsc.bitcast(vi << 16, jnp.float32)
                            od = plsc.bitcast(vi & himask, jnp.float32)
                        else:
                            ev = rows[b, e, pl.ds(32 * hh, 16)]
                            od = rows[b, e, pl.ds(32 * hh + 16, 16)]
                        p0.append(th[hh] * ev)
                        p1.append(th[hh] * od)
                    msg[b, e, pl.ds(0, 16)] = _tree8(p0) / s
                    msg[b, e, pl.ds(16, 16)] = _tree8(p1) / s
            return 0

        lax.fori_loop(0, chunk // 4, _quad, 0)

    _issue_idx(0, 0)
    _issue_idx(1, 1)
    _wait_idx(0)
    _issue_gathers(0, 0)

    def _body(g, k):
        c = 4 * g + k
        b = k % 2
        bn = 1 - b

        @pl.when(c + 1 < nchunk)
        def _():
            _wait_idx((k + 1) % 4)
            _issue_gathers(bn, (k + 1) % 4)

        _wait_gathers(b, k)

        @pl.when(c >= 2)
        def _():
            _wait_scatter(b, (k + 2) % 4)

        @pl.when(c + 2 < nchunk)
        def _():
            _issue_idx(c + 2, (k + 2) % 4)

        _compute(b)
        _issue_scatter(b, k)

    def _quad_body(g, _):
        for k in range(4):
            _body(g, k)
        return 0

    lax.fori_loop(0, nchunk // 4, _quad_body, 0)
    _wait_scatter(0, (nchunk - 2) % 4)
    _wait_scatter(1, (nchunk - 1) % 4)
    plsc.subcore_barrier()

    def _copyout(z, _):
        r0 = sid * _ROWS_PER_TILE + z * ZCH
        pltpu.sync_copy(acc.at[pl.ds(r0, ZCH)],
                        out_hbm.at[pl.ds(cid * ACC_ROWS + r0, ZCH)])
        return 0

    lax.fori_loop(0, ZN, _copyout, 0)

  return _edge_sc


_edge_sc_f32 = _make_edge_sc(jnp.float32, 32, 784)
_edge_sc_w = _make_edge_sc(jnp.bfloat16, 32, 784, wide=True)


@functools.partial(
    pl.kernel,
    out_type=jax.ShapeDtypeStruct((SC_CORES * ACC_ROWS, 16), jnp.float32),
    mesh=_SC_MESH,
    scratch_types=[
        pltpu.VMEM((2, DCH), jnp.int32),
        pltpu.VMEM((DCH, 16), jnp.float32),
        pltpu.VMEM((ZCH, 16), jnp.float32),
        pltpu.VMEM_SHARED((ACC_ROWS, 16), jnp.float32),
        pltpu.SemaphoreType.DMA,
        pltpu.SemaphoreType.DMA,
    ],
    compiler_params=pltpu.CompilerParams(use_tc_tiling_on_sc=False),
)
def _deg_sc(dst_hbm, out_hbm, didx, ones, zrows, acc, semd0, semd1):
    cid = lax.axis_index("c")
    sid = lax.axis_index("s")
    wid = sid * SC_CORES + cid
    iota = lax.iota(jnp.int32, 16)
    onev = jnp.where(iota == 0, 1.0, 0.0).astype(jnp.float32)
    zv = jnp.zeros((16,), jnp.float32)
    semd = (semd0, semd1)

    def _fill(i, _):
        ones[i, pl.ds(0, 16)] = onev
        return 0

    lax.fori_loop(0, DCH, _fill, 0)

    def _fillz(i, _):
        zrows[i, pl.ds(0, 16)] = zv
        return 0

    lax.fori_loop(0, ZCH, _fillz, 0)

    def _zacc(z, _):
        pltpu.sync_copy(zrows, acc.at[pl.ds(sid * _ROWS_PER_TILE + z * ZCH,
                                            ZCH)])
        return 0

    lax.fori_loop(0, ZN, _zacc, 0)
    plsc.subcore_barrier()

    tile_base = wid * (DNCH * DCH)

    def _dissue(c, b):
        pltpu.async_copy(dst_hbm.at[pl.ds(tile_base + c * DCH, DCH)],
                         didx.at[b], semd[b])

    def _dwait(b):
        pltpu.make_async_copy(dst_hbm.at[pl.ds(tile_base, DCH)], didx.at[b],
                              semd[b]).wait()

    _dissue(0, 0)
    _dissue(1, 1)

    def _dbody(c, b):
        _dwait(b)
        pltpu.sync_copy(ones, acc.at[didx.at[b]], add=True)

        @pl.when(c + 2 < DNCH)
        def _():
            _dissue(c + 2, b)

    def _dpair(g, _):
        _dbody(2 * g, 0)
        _dbody(2 * g + 1, 1)
        return 0

    lax.fori_loop(0, DNCH // 2, _dpair, 0)
    plsc.subcore_barrier()

    def _copyout(z, _):
        r0 = sid * _ROWS_PER_TILE + z * ZCH
        pltpu.sync_copy(acc.at[pl.ds(r0, ZCH)],
                        out_hbm.at[pl.ds(cid * ACC_ROWS + r0, ZCH)])
        return 0

    lax.fori_loop(0, ZN, _copyout, 0)


def kernel(x, edge_index, batch, fc0_w, fc0_b, conv1_weight, conv1_u, conv1_c,
           conv1_bias, conv2_weight, conv2_u, conv2_c, conv2_bias, fc1_w,
           fc1_b):
    f32 = jnp.float32
    q01 = jax.nn.softmax(conv1_c)[None, :]
    q02 = jax.nn.softmax(conv2_c)[None, :]
    cc1 = jnp.concatenate([conv1_c, jnp.full((8,), -1e30, f32)])
    cc2 = jnp.concatenate([conv2_c, jnp.full((8,), -1e30, f32)])


    src = jnp.concatenate(
        [edge_index[0], jnp.zeros((EPAD - E,), jnp.int32)])
    dst = jnp.concatenate(
        [edge_index[1], jnp.full((EPAD - E,), DUMMY_DST, jnp.int32)])
    e2 = jnp.stack([src, dst])

    hw1, hu1, sm1 = _dense1(x, fc0_w, fc0_b[None, :], conv1_weight, conv1_u,
                            q01)
    hu1p = jnp.zeros((HU_ROWS, 16), f32).at[0:N].set(hu1)
    dega = _deg_sc(dst).reshape(2, ACC_ROWS, 16)
    acc1 = _edge_sc_f32(e2, hu1p, hw1, cc1).reshape(2, ACC_ROWS, 32)

    tw, hu2, sm2, deg = _dense2(acc1, dega, sm1, conv1_bias[None, :],
                                conv2_weight, conv2_u, q02)
    hu2p = jnp.zeros((HU_ROWS, 16), f32).at[0:N].set(hu2)
    accF = _edge_sc_w(e2, hu2p, tw, cc2).reshape(2, ACC_ROWS, 64)

    oh = (batch[:, None] == jnp.arange(16, dtype=jnp.int32)[None, :]
          ).astype(f32)
    out16 = _dense3(accF, sm2, deg, conv2_bias[None, :], oh,
                    jnp.zeros((64, 16), f32).at[:, 0:10].set(fc1_w),
                    jnp.zeros((1, 16), f32).at[0, 0:10].set(fc1_b))
    return out16[:, 0:10]
